# baseline (speedup 1.0000x reference)
import functools

import jax
import jax.numpy as jnp
import numpy as np
from jax import lax
from jax.experimental import pallas as pl
from jax.experimental.pallas import tpu as pltpu
from jax.experimental.pallas import tpu_sc as plsc

S, D, H, E, F = 2048, 768, 12, 64, 64
DH = D // H
NB = S // 128
ND = 2 * NB - 1
NP = 2 * S
NT = 96
NW = 32

_INTERPRET = False


def _pc(body, grid, in_specs, out_specs, out_shape, scratch_shapes=()):
    return pl.pallas_call(
        body,
        grid=grid,
        in_specs=in_specs,
        out_specs=out_specs,
        out_shape=out_shape,
        scratch_shapes=list(scratch_shapes),
        interpret=_INTERPRET,
    )


def _ln(x, g, b):
    m = jnp.mean(x, axis=-1, keepdims=True)
    v = jnp.mean((x - m) ** 2, axis=-1, keepdims=True)
    return (x - m) * jax.lax.rsqrt(v + 1e-5) * g + b


def _k1_body(src_ref, g_ref, b_ref, wq_ref, wk_ref, wv_ref, q_ref, k_ref, v_ref):
    x2 = _ln(src_ref[...], g_ref[...], b_ref[...]).astype(jnp.bfloat16)
    q = lax.dot(x2, wq_ref[...], preferred_element_type=jnp.float32) * 0.125
    k = lax.dot(x2, wk_ref[...], preferred_element_type=jnp.float32)
    v = lax.dot(x2, wv_ref[...], preferred_element_type=jnp.float32)
    qb, kb, vb = q.astype(jnp.bfloat16), k.astype(jnp.bfloat16), v.astype(jnp.bfloat16)
    ones = jnp.ones((128, 1), jnp.bfloat16)
    zeros = jnp.zeros((128, DH - 1), jnp.bfloat16)
    for h in range(H):
        sl = slice(h * DH, (h + 1) * DH)
        q_ref[h] = qb[:, sl]
        k_ref[h] = kb[:, sl]
        v_ref[h] = jnp.concatenate([vb[:, sl], ones, zeros], axis=1)


def _k1(src, ln1_g, ln1_b, wq, wk, wv):
    spec_w = pl.BlockSpec((D, D), lambda i: (0, 0))
    spec_v = pl.BlockSpec((1, D), lambda i: (0, 0))
    out_spec = pl.BlockSpec((H, 128, DH), lambda i: (0, i, 0))
    out_spec_v = pl.BlockSpec((H, 128, 2 * DH), lambda i: (0, i, 0))
    return _pc(
        _k1_body,
        grid=(NB,),
        in_specs=[pl.BlockSpec((128, D), lambda i: (i, 0)), spec_v, spec_v,
                  spec_w, spec_w, spec_w],
        out_specs=[out_spec, out_spec, out_spec_v],
        out_shape=[jax.ShapeDtypeStruct((H, S, DH), jnp.bfloat16),
                   jax.ShapeDtypeStruct((H, S, DH), jnp.bfloat16),
                   jax.ShapeDtypeStruct((H, S, 2 * DH), jnp.bfloat16)],
    )(src, ln1_g.reshape(1, D), ln1_b.reshape(1, D),
      wq.astype(jnp.bfloat16), wk.astype(jnp.bfloat16), wv.astype(jnp.bfloat16))


def _k2_body(rb_ref, e_ref, out_ref):
    rb = rb_ref[...]
    sm = jnp.concatenate(
        [rb[:, 128 * d:128 * d + 256] for d in range(ND)], axis=0
    ).astype(jnp.bfloat16)
    for nc in range(4):
        sl = slice(nc * 4096, (nc + 1) * 4096)
        mm = lax.dot(sm, e_ref[:, sl], preferred_element_type=jnp.float32)
        out_ref[:, sl] = mm.astype(jnp.bfloat16)


def _k2(rel_bias):
    rb = jnp.pad(rel_bias, ((0, 0), (0, 4096 - (2 * S - 1))))
    ab = np.arange(128 * 128)
    a, b = ab // 128, ab % 128
    c = np.arange(256)
    e_mat = (c[:, None] == (b - a + 127)[None, :]).astype(np.float32)
    e_mat = jnp.asarray(e_mat, dtype=jnp.bfloat16)
    t3 = _pc(
        _k2_body,
        grid=(1,),
        in_specs=[pl.BlockSpec((H, 4096), lambda i: (0, 0)),
                  pl.BlockSpec((256, 128 * 128), lambda i: (0, 0))],
        out_specs=pl.BlockSpec((ND * H, 128 * 128), lambda i: (0, 0)),
        out_shape=jax.ShapeDtypeStruct((ND * H, 128 * 128), jnp.bfloat16),
    )(rb, e_mat)
    return t3.reshape(ND, H, 128, 128)


def _k3_body(q_ref, k_ref, v_ref, t_ref, o_ref):
    h = pl.program_id(0)
    i = pl.program_id(1)
    q = q_ref[0]
    k = k_ref[0]
    s = lax.dot_general(q, k, (((1,), (1,)), ((), ())),
                        preferred_element_type=jnp.float32)
    patt = jnp.concatenate(
        [t_ref[j - i + (NB - 1), h].astype(jnp.float32) for j in range(NB)], axis=1)
    s = s + patt
    m = jnp.max(s, axis=1, keepdims=True)
    p = jnp.exp((s - m).astype(jnp.bfloat16))
    o2 = lax.dot(p, v_ref[0], preferred_element_type=jnp.float32)
    o = o2[:, :DH] * (1.0 / o2[:, DH:DH + 1])
    o_ref[0] = o.astype(jnp.bfloat16)


def _k3(q, k, v, t4):
    return _pc(
        _k3_body,
        grid=(H, NB),
        in_specs=[pl.BlockSpec((1, 128, DH), lambda h, i: (h, i, 0)),
                  pl.BlockSpec((1, S, DH), lambda h, i: (h, 0, 0)),
                  pl.BlockSpec((1, S, 2 * DH), lambda h, i: (h, 0, 0)),
                  pl.BlockSpec((ND, H, 128, 128), lambda h, i: (0, 0, 0, 0))],
        out_specs=pl.BlockSpec((1, 128, DH), lambda h, i: (h, i, 0)),
        out_shape=jax.ShapeDtypeStruct((H, S, DH), jnp.bfloat16),
    )(q, k, v, t4)


def _k45_body(att_ref, wo_ref, src_ref, g_ref, b_ref, es_ref,
              src2_ref, x3_ref, gates_ref, idx_ref):
    cat = jnp.concatenate([att_ref[h] for h in range(H)], axis=1)
    o = lax.dot(cat, wo_ref[...], preferred_element_type=jnp.float32)
    src2 = src_ref[...] + o
    src2_ref[...] = src2
    x3 = _ln(src2, g_ref[...], b_ref[...])
    x3_ref[...] = x3.astype(jnp.bfloat16)
    logits = lax.dot(x3, es_ref[...], preferred_element_type=jnp.float32)
    sel = jax.nn.sigmoid(logits)
    iota = lax.broadcasted_iota(jnp.int32, sel.shape, 1)
    m1 = jnp.max(sel, axis=1, keepdims=True)
    i1 = jnp.min(jnp.where(sel == m1, iota, E), axis=1, keepdims=True)
    masked = jnp.where(iota == i1, -1.0, sel)
    m2 = jnp.max(masked, axis=1, keepdims=True)
    i2 = jnp.min(jnp.where(masked == m2, iota, E), axis=1, keepdims=True)
    gates_ref[...] = jnp.concatenate([m1, m2], axis=1)
    idx_ref[...] = jnp.concatenate([i1, i2], axis=1)


def _k45(att, wo, src, ln2_g, ln2_b, expert_sel):
    spec_v = pl.BlockSpec((1, D), lambda i: (0, 0))
    return _pc(
        _k45_body,
        grid=(NB,),
        in_specs=[pl.BlockSpec((H, 128, DH), lambda i: (0, i, 0)),
                  pl.BlockSpec((D, D), lambda i: (0, 0)),
                  pl.BlockSpec((128, D), lambda i: (i, 0)),
                  spec_v, spec_v,
                  pl.BlockSpec((D, E), lambda i: (0, 0))],
        out_specs=[pl.BlockSpec((128, D), lambda i: (i, 0)),
                   pl.BlockSpec((128, D), lambda i: (i, 0)),
                   pl.BlockSpec((128, 2), lambda i: (i, 0)),
                   pl.BlockSpec((128, 2), lambda i: (i, 0))],
        out_shape=[jax.ShapeDtypeStruct((S, D), jnp.float32),
                   jax.ShapeDtypeStruct((S, D), jnp.bfloat16),
                   jax.ShapeDtypeStruct((S, 2), jnp.float32),
                   jax.ShapeDtypeStruct((S, 2), jnp.int32)],
    )(att, wo.astype(jnp.bfloat16), src, ln2_g.reshape(1, D),
      ln2_b.reshape(1, D), expert_sel)


def _k6s_body(idx_ref, lt_ref, sl_ref, pos_ref, eot_ref):
    def pass1(pb, cnt):
        idxb = idx_ref[pl.ds(pb * 128, 128), :]
        oh = (idxb == lax.broadcasted_iota(jnp.int32, (128, E), 1))
        ohb = oh.astype(jnp.bfloat16)
        cums = lax.dot(lt_ref[...], ohb, preferred_element_type=jnp.float32)
        rank = jnp.sum(oh.astype(jnp.float32) * (cums + cnt), axis=1,
                       keepdims=True)
        pos_ref[pl.ds(pb * 128, 128), :] = rank.astype(jnp.int32)
        return cnt + jnp.sum(oh.astype(jnp.float32), axis=0, keepdims=True)

    cnt = lax.fori_loop(0, NP // 128, pass1, jnp.zeros((1, E), jnp.float32))
    ntiles = jnp.ceil(cnt * (1.0 / 128.0))
    tilestart = lax.dot(ntiles.astype(jnp.bfloat16), sl_ref[...],
                        preferred_element_type=jnp.float32)
    startpad = tilestart * 128.0

    def pass2(pb, _):
        idxb = idx_ref[pl.ds(pb * 128, 128), :]
        oh = (idxb == lax.broadcasted_iota(jnp.int32, (128, E), 1))
        add = jnp.sum(oh.astype(jnp.float32) * startpad, axis=1,
                      keepdims=True).astype(jnp.int32)
        pos_ref[pl.ds(pb * 128, 128), :] = pos_ref[pl.ds(pb * 128, 128), :] + add
        return 0

    lax.fori_loop(0, NP // 128, pass2, 0)
    ts_i = tilestart.astype(jnp.int32)
    tio = lax.broadcasted_iota(jnp.int32, (NT, E), 0)
    eot_ref[...] = jnp.sum((tio >= ts_i).astype(jnp.int32), axis=1,
                           keepdims=True) - 1


def _k6_sort(idxp):
    lt = jnp.asarray(np.tril(np.ones((128, 128), np.float32), -1),
                     dtype=jnp.bfloat16)
    sl = jnp.asarray(np.triu(np.ones((E, E), np.float32), 1),
                     dtype=jnp.bfloat16)
    return _pc(
        _k6s_body,
        grid=(1,),
        in_specs=[pl.BlockSpec((NP, 1), lambda i: (0, 0)),
                  pl.BlockSpec((128, 128), lambda i: (0, 0)),
                  pl.BlockSpec((E, E), lambda i: (0, 0))],
        out_specs=[pl.BlockSpec((NP, 1), lambda i: (0, 0)),
                   pl.BlockSpec((NT, 1), lambda i: (0, 0))],
        out_shape=[jax.ShapeDtypeStruct((NP, 1), jnp.int32),
                   jax.ShapeDtypeStruct((NT, 1), jnp.int32)],
    )(idxp, lt, sl)


def _k7_body(eot_sref, xg_ref, k_ref, v_ref, yg_ref):
    x = xg_ref[...]
    hid = jax.nn.relu(lax.dot(x, k_ref[0], preferred_element_type=jnp.float32))
    yg_ref[...] = lax.dot(hid.astype(jnp.bfloat16), v_ref[0],
                          preferred_element_type=jnp.float32)


def _k7_group(xg, keys_bf, values_bf, eot_flat):
    grid_spec = pltpu.PrefetchScalarGridSpec(
        num_scalar_prefetch=1,
        grid=(NT,),
        in_specs=[pl.BlockSpec((128, D), lambda t, e: (t, 0)),
                  pl.BlockSpec((1, D, F), lambda t, e: (e[t], 0, 0)),
                  pl.BlockSpec((1, F, D), lambda t, e: (e[t], 0, 0))],
        out_specs=pl.BlockSpec((128, D), lambda t, e: (t, 0)),
    )
    return pl.pallas_call(
        _k7_body,
        grid_spec=grid_spec,
        out_shape=jax.ShapeDtypeStruct((NT * 128, D), jnp.float32),
        interpret=_INTERPRET,
    )(eot_flat, xg, keys_bf, values_bf)


def _sc_mesh():
    return plsc.VectorSubcoreMesh(core_axis_name="c", subcore_axis_name="s",
                                  num_cores=2)


def _k8_dispatch(x3i, pos2, tok2):
    @functools.partial(
        pl.kernel, mesh=_sc_mesh(),
        out_type=jax.ShapeDtypeStruct((NT * 128, D // 2), jnp.int32),
        scratch_types=[pltpu.VMEM((2, 64), jnp.int32),
                       pltpu.VMEM((2, 64), jnp.int32),
                       pltpu.VMEM((64, D // 2), jnp.int32),
                       pltpu.VMEM((64, D // 2), jnp.int32),
                       pltpu.SemaphoreType.DMA,
                       pltpu.SemaphoreType.DMA],
    )
    def k(x3_hbm, pos_hbm, tok_hbm, xg_hbm, tok_v, pos_v, rows_a, rows_b,
          sem_g, sem_s):
        wid = lax.axis_index("s") * 2 + lax.axis_index("c")
        pltpu.sync_copy(tok_hbm.at[pl.ds(wid * 2, 2)], tok_v)
        pltpu.sync_copy(pos_hbm.at[pl.ds(wid * 2, 2)], pos_v)
        pltpu.async_copy(x3_hbm.at[tok_v.at[0]], rows_a, sem_g).wait()
        s0 = pltpu.async_copy(rows_a, xg_hbm.at[pos_v.at[0]], sem_s)
        pltpu.async_copy(x3_hbm.at[tok_v.at[1]], rows_b, sem_g).wait()
        s1 = pltpu.async_copy(rows_b, xg_hbm.at[pos_v.at[1]], sem_s)
        s0.wait()
        s1.wait()

    return k(x3i, pos2, tok2)


def _k9_combine(yg, p0, p1, g0, g1, src2):
    @functools.partial(
        pl.kernel, mesh=_sc_mesh(),
        out_type=jax.ShapeDtypeStruct((S, D), jnp.float32),
        scratch_types=[pltpu.VMEM((4, 16), jnp.int32),
                       pltpu.VMEM((4, 16), jnp.int32),
                       pltpu.VMEM((4, 16), jnp.float32),
                       pltpu.VMEM((4, 16), jnp.float32),
                       pltpu.VMEM((16, D), jnp.float32),
                       pltpu.VMEM((16, D), jnp.float32),
                       pltpu.VMEM((16, D), jnp.float32),
                       pltpu.SemaphoreType.DMA],
    )
    def k(yg_hbm, p0_hbm, p1_hbm, g0_hbm, g1_hbm, src_hbm, out_hbm,
          p0v, p1v, g0v, g1v, r0v, r1v, sv, sem):
        wid = lax.axis_index("s") * 2 + lax.axis_index("c")
        pltpu.sync_copy(p0_hbm.at[pl.ds(wid * 4, 4)], p0v)
        pltpu.sync_copy(p1_hbm.at[pl.ds(wid * 4, 4)], p1v)
        pltpu.sync_copy(g0_hbm.at[pl.ds(wid * 4, 4)], g0v)
        pltpu.sync_copy(g1_hbm.at[pl.ds(wid * 4, 4)], g1v)
        for c in range(4):
            c0 = pltpu.async_copy(yg_hbm.at[p0v.at[c]], r0v, sem)
            c1 = pltpu.async_copy(yg_hbm.at[p1v.at[c]], r1v, sem)
            base = wid * 64 + c * 16
            pltpu.sync_copy(src_hbm.at[pl.ds(base, 16)], sv)
            g0row = g0v[c, :]
            g1row = g1v[c, :]
            ga = [jnp.full((16,), g0row[i], jnp.float32) for i in range(16)]
            gb = [jnp.full((16,), g1row[i], jnp.float32) for i in range(16)]
            c0.wait()
            c1.wait()

            def col(j, _2):
                sl = pl.ds(j * 16, 16)
                for i in range(16):
                    sv[i, sl] = sv[i, sl] + ga[i] * r0v[i, sl] + gb[i] * r1v[i, sl]
                return 0

            lax.fori_loop(0, D // 16, col, 0)
            pltpu.sync_copy(sv, out_hbm.at[pl.ds(base, 16)])

    return k(yg, p0, p1, g0, g1, src2)


_TOK2 = np.repeat(np.arange(S, dtype=np.int32), 2).reshape(NW * 2, 64)


def kernel(src, ln1_g, ln1_b, ln2_g, ln2_b, Wq, Wk, Wv, Wo, rel_bias,
           expert_sel, keys, values):
    src2d = src.reshape(S, D)
    q, k, v = _k1(src2d, ln1_g, ln1_b, Wq, Wk, Wv)
    t4 = _k2(rel_bias)
    att = _k3(q, k, v, t4)
    src2, x3bf, gates2, idx2 = _k45(att, Wo, src2d, ln2_g, ln2_b, expert_sel)
    pos, eot = _k6_sort(idx2.reshape(NP, 1))
    x3i = lax.bitcast_convert_type(x3bf.reshape(S, D // 2, 2), jnp.int32)
    xgi = _k8_dispatch(x3i, pos.reshape(NW * 2, 64), jnp.asarray(_TOK2))
    xg = lax.bitcast_convert_type(xgi, jnp.bfloat16).reshape(NT * 128, D)
    yg = _k7_group(xg, keys.astype(jnp.bfloat16), values.astype(jnp.bfloat16),
                   eot.reshape(NT))
    posT = pos.reshape(S, 2)
    out = _k9_combine(yg,
                      posT[:, 0].reshape(NW * 4, 16),
                      posT[:, 1].reshape(NW * 4, 16),
                      gates2[:, 0].reshape(NW * 4, 16),
                      gates2[:, 1].reshape(NW * 4, 16),
                      src2)
    return out.reshape(1, S, D)


# speedup vs baseline: 1.5180x; 1.5180x over previous
import functools

import jax
import jax.numpy as jnp
import numpy as np
from jax import lax
from jax.experimental import pallas as pl
from jax.experimental.pallas import tpu as pltpu
from jax.experimental.pallas import tpu_sc as plsc

S, D, H, E, F = 2048, 768, 12, 64, 64
DH = D // H
NB = S // 128
ND = 2 * NB - 1
NP = 2 * S
NT = 96
NW = 32

_INTERPRET = False


def _pc(body, grid, in_specs, out_specs, out_shape, scratch_shapes=()):
    return pl.pallas_call(
        body,
        grid=grid,
        in_specs=in_specs,
        out_specs=out_specs,
        out_shape=out_shape,
        scratch_shapes=list(scratch_shapes),
        interpret=_INTERPRET,
    )


def _ln(x, g, b):
    m = jnp.mean(x, axis=-1, keepdims=True)
    v = jnp.mean((x - m) ** 2, axis=-1, keepdims=True)
    return (x - m) * jax.lax.rsqrt(v + 1e-5) * g + b


def _k1_body(src_ref, g_ref, b_ref, wq_ref, wk_ref, wv_ref, q_ref, k_ref, v_ref):
    x2 = _ln(src_ref[...], g_ref[...], b_ref[...]).astype(jnp.bfloat16)
    q = lax.dot(x2, wq_ref[...], preferred_element_type=jnp.float32) * 0.125
    k = lax.dot(x2, wk_ref[...], preferred_element_type=jnp.float32)
    v = lax.dot(x2, wv_ref[...], preferred_element_type=jnp.float32)
    qb, kb, vb = q.astype(jnp.bfloat16), k.astype(jnp.bfloat16), v.astype(jnp.bfloat16)
    ones = jnp.ones((128, 1), jnp.bfloat16)
    zeros = jnp.zeros((128, DH - 1), jnp.bfloat16)
    for h in range(H):
        sl = slice(h * DH, (h + 1) * DH)
        q_ref[h] = qb[:, sl]
        k_ref[h] = kb[:, sl]
        v_ref[h] = jnp.concatenate([vb[:, sl], ones, zeros], axis=1)


def _k1(src, ln1_g, ln1_b, wq, wk, wv):
    spec_w = pl.BlockSpec((D, D), lambda i: (0, 0))
    spec_v = pl.BlockSpec((1, D), lambda i: (0, 0))
    out_spec = pl.BlockSpec((H, 128, DH), lambda i: (0, i, 0))
    out_spec_v = pl.BlockSpec((H, 128, 2 * DH), lambda i: (0, i, 0))
    return _pc(
        _k1_body,
        grid=(NB,),
        in_specs=[pl.BlockSpec((128, D), lambda i: (i, 0)), spec_v, spec_v,
                  spec_w, spec_w, spec_w],
        out_specs=[out_spec, out_spec, out_spec_v],
        out_shape=[jax.ShapeDtypeStruct((H, S, DH), jnp.bfloat16),
                   jax.ShapeDtypeStruct((H, S, DH), jnp.bfloat16),
                   jax.ShapeDtypeStruct((H, S, 2 * DH), jnp.bfloat16)],
    )(src, ln1_g.reshape(1, D), ln1_b.reshape(1, D),
      wq.astype(jnp.bfloat16), wk.astype(jnp.bfloat16), wv.astype(jnp.bfloat16))


def _k2_body(rb_ref, e_ref, out_ref):
    rb = rb_ref[...]
    sm = jnp.concatenate(
        [rb[:, 128 * d:128 * d + 256] for d in range(ND)], axis=0
    ).astype(jnp.bfloat16)
    for nc in range(4):
        sl = slice(nc * 4096, (nc + 1) * 4096)
        mm = lax.dot(sm, e_ref[:, sl], preferred_element_type=jnp.float32)
        out_ref[:, sl] = mm.astype(jnp.bfloat16)


def _k2(rel_bias):
    rb = jnp.pad(rel_bias, ((0, 0), (0, 4096 - (2 * S - 1))))
    ab = np.arange(128 * 128)
    a, b = ab // 128, ab % 128
    c = np.arange(256)
    e_mat = (c[:, None] == (b - a + 127)[None, :]).astype(np.float32)
    e_mat = jnp.asarray(e_mat, dtype=jnp.bfloat16)
    t3 = _pc(
        _k2_body,
        grid=(1,),
        in_specs=[pl.BlockSpec((H, 4096), lambda i: (0, 0)),
                  pl.BlockSpec((256, 128 * 128), lambda i: (0, 0))],
        out_specs=pl.BlockSpec((ND * H, 128 * 128), lambda i: (0, 0)),
        out_shape=jax.ShapeDtypeStruct((ND * H, 128 * 128), jnp.bfloat16),
    )(rb, e_mat)
    return t3.reshape(ND, H, 128, 128)


def _k3_body(q_ref, k_ref, v_ref, t_ref, o_ref):
    h = pl.program_id(0)
    i = pl.program_id(1)
    q = q_ref[0]
    k = k_ref[0]
    s = lax.dot_general(q, k, (((1,), (1,)), ((), ())),
                        preferred_element_type=jnp.float32)
    patt = jnp.concatenate(
        [t_ref[j - i + (NB - 1), h].astype(jnp.float32) for j in range(NB)], axis=1)
    s = s + patt
    m = jnp.max(s, axis=1, keepdims=True)
    p = jnp.exp((s - m).astype(jnp.bfloat16))
    o2 = lax.dot(p, v_ref[0], preferred_element_type=jnp.float32)
    o = o2[:, :DH] * (1.0 / o2[:, DH:DH + 1])
    o_ref[0] = o.astype(jnp.bfloat16)


def _k3(q, k, v, t4):
    return _pc(
        _k3_body,
        grid=(H, NB),
        in_specs=[pl.BlockSpec((1, 128, DH), lambda h, i: (h, i, 0)),
                  pl.BlockSpec((1, S, DH), lambda h, i: (h, 0, 0)),
                  pl.BlockSpec((1, S, 2 * DH), lambda h, i: (h, 0, 0)),
                  pl.BlockSpec((ND, H, 128, 128), lambda h, i: (0, 0, 0, 0))],
        out_specs=pl.BlockSpec((1, 128, DH), lambda h, i: (h, i, 0)),
        out_shape=jax.ShapeDtypeStruct((H, S, DH), jnp.bfloat16),
    )(q, k, v, t4)


def _k45_body(att_ref, wo_ref, src_ref, g_ref, b_ref, es_ref,
              src2_ref, x3_ref, g0_ref, g1_ref, idx_ref):
    cat = jnp.concatenate([att_ref[h] for h in range(H)], axis=1)
    o = lax.dot(cat, wo_ref[...], preferred_element_type=jnp.float32)
    src2 = src_ref[...] + o
    src2_ref[...] = src2
    x3 = _ln(src2, g_ref[...], b_ref[...])
    x3_ref[...] = x3
    logits = lax.dot(x3, es_ref[...], preferred_element_type=jnp.float32)
    sel = jax.nn.sigmoid(logits)
    iota = lax.broadcasted_iota(jnp.int32, sel.shape, 1)
    m1 = jnp.max(sel, axis=1, keepdims=True)
    i1 = jnp.min(jnp.where(sel == m1, iota, E), axis=1, keepdims=True)
    masked = jnp.where(iota == i1, -1.0, sel)
    m2 = jnp.max(masked, axis=1, keepdims=True)
    i2 = jnp.min(jnp.where(masked == m2, iota, E), axis=1, keepdims=True)
    g0_ref[...] = m1
    g1_ref[...] = m2
    idx_ref[...] = jnp.concatenate([i1, i2], axis=1)


def _k45(att, wo, src, ln2_g, ln2_b, expert_sel):
    spec_v = pl.BlockSpec((1, D), lambda i: (0, 0))
    spec_1 = pl.BlockSpec((128, 1), lambda i: (i, 0))
    return _pc(
        _k45_body,
        grid=(NB,),
        in_specs=[pl.BlockSpec((H, 128, DH), lambda i: (0, i, 0)),
                  pl.BlockSpec((D, D), lambda i: (0, 0)),
                  pl.BlockSpec((128, D), lambda i: (i, 0)),
                  spec_v, spec_v,
                  pl.BlockSpec((D, E), lambda i: (0, 0))],
        out_specs=[pl.BlockSpec((128, D), lambda i: (i, 0)),
                   pl.BlockSpec((128, D), lambda i: (i, 0)),
                   spec_1, spec_1,
                   pl.BlockSpec((128, 2), lambda i: (i, 0))],
        out_shape=[jax.ShapeDtypeStruct((S, D), jnp.float32),
                   jax.ShapeDtypeStruct((S, D), jnp.float32),
                   jax.ShapeDtypeStruct((S, 1), jnp.float32),
                   jax.ShapeDtypeStruct((S, 1), jnp.float32),
                   jax.ShapeDtypeStruct((S, 2), jnp.int32)],
    )(att, wo.astype(jnp.bfloat16), src, ln2_g.reshape(1, D),
      ln2_b.reshape(1, D), expert_sel)


def _k6s_body(idx_ref, lt_ref, sl_ref, p0_ref, p1_ref, eot_ref):
    def ohs_of(pb):
        idxb = idx_ref[pl.ds(pb * 128, 128), :]
        iota = lax.broadcasted_iota(jnp.int32, (128, E), 1)
        oh0 = (idxb[:, 0:1] == iota).astype(jnp.float32)
        oh1 = (idxb[:, 1:2] == iota).astype(jnp.float32)
        return oh0, oh1

    def pass1(pb, cnt):
        oh0, oh1 = ohs_of(pb)
        ohs = oh0 + oh1
        cums = lax.dot(lt_ref[...], ohs.astype(jnp.bfloat16),
                       preferred_element_type=jnp.float32)
        r0 = jnp.sum(oh0 * (cums + cnt), axis=1, keepdims=True)
        r1 = jnp.sum(oh1 * (cums + cnt + oh0), axis=1, keepdims=True)
        p0_ref[pl.ds(pb * 128, 128), :] = r0.astype(jnp.int32)
        p1_ref[pl.ds(pb * 128, 128), :] = r1.astype(jnp.int32)
        return cnt + jnp.sum(ohs, axis=0, keepdims=True)

    cnt = lax.fori_loop(0, S // 128, pass1, jnp.zeros((1, E), jnp.float32))
    ntiles = jnp.ceil(cnt * (1.0 / 128.0))
    tilestart = lax.dot(ntiles.astype(jnp.bfloat16), sl_ref[...],
                        preferred_element_type=jnp.float32)
    startpad = tilestart * 128.0

    def pass2(pb, _):
        oh0, oh1 = ohs_of(pb)
        a0 = jnp.sum(oh0 * startpad, axis=1, keepdims=True).astype(jnp.int32)
        a1 = jnp.sum(oh1 * startpad, axis=1, keepdims=True).astype(jnp.int32)
        p0_ref[pl.ds(pb * 128, 128), :] = p0_ref[pl.ds(pb * 128, 128), :] + a0
        p1_ref[pl.ds(pb * 128, 128), :] = p1_ref[pl.ds(pb * 128, 128), :] + a1
        return 0

    lax.fori_loop(0, S // 128, pass2, 0)
    ts_i = tilestart.astype(jnp.int32)
    tio = lax.broadcasted_iota(jnp.int32, (NT, E), 0)
    eot_ref[...] = jnp.sum((tio >= ts_i).astype(jnp.int32), axis=1,
                           keepdims=True) - 1


def _k6_sort(idx2):
    lt = jnp.asarray(np.tril(np.ones((128, 128), np.float32), -1),
                     dtype=jnp.bfloat16)
    sl = jnp.asarray(np.triu(np.ones((E, E), np.float32), 1),
                     dtype=jnp.bfloat16)
    return _pc(
        _k6s_body,
        grid=(1,),
        in_specs=[pl.BlockSpec((S, 2), lambda i: (0, 0)),
                  pl.BlockSpec((128, 128), lambda i: (0, 0)),
                  pl.BlockSpec((E, E), lambda i: (0, 0))],
        out_specs=[pl.BlockSpec((S, 1), lambda i: (0, 0)),
                   pl.BlockSpec((S, 1), lambda i: (0, 0)),
                   pl.BlockSpec((NT, 1), lambda i: (0, 0))],
        out_shape=[jax.ShapeDtypeStruct((S, 1), jnp.int32),
                   jax.ShapeDtypeStruct((S, 1), jnp.int32),
                   jax.ShapeDtypeStruct((NT, 1), jnp.int32)],
    )(idx2, lt, sl)


def _k7_body(eot_sref, xg_ref, k_ref, v_ref, yg_ref):
    x = xg_ref[...].astype(jnp.bfloat16)
    hid = jax.nn.relu(lax.dot(x, k_ref[0], preferred_element_type=jnp.float32))
    yg_ref[...] = lax.dot(hid.astype(jnp.bfloat16), v_ref[0],
                          preferred_element_type=jnp.float32)


def _k7_group(xg, keys_bf, values_bf, eot_flat):
    grid_spec = pltpu.PrefetchScalarGridSpec(
        num_scalar_prefetch=1,
        grid=(NT,),
        in_specs=[pl.BlockSpec((128, D), lambda t, e: (t, 0)),
                  pl.BlockSpec((1, D, F), lambda t, e: (e[t], 0, 0)),
                  pl.BlockSpec((1, F, D), lambda t, e: (e[t], 0, 0))],
        out_specs=pl.BlockSpec((128, D), lambda t, e: (t, 0)),
    )
    return pl.pallas_call(
        _k7_body,
        grid_spec=grid_spec,
        out_shape=jax.ShapeDtypeStruct((NT * 128, D), jnp.float32),
        interpret=_INTERPRET,
    )(eot_flat, xg, keys_bf, values_bf)


def _sc_mesh():
    return plsc.VectorSubcoreMesh(core_axis_name="c", subcore_axis_name="s",
                                  num_cores=2)


def _k8_dispatch(x3, p0, p1):
    @functools.partial(
        pl.kernel, mesh=_sc_mesh(),
        out_type=jax.ShapeDtypeStruct((NT * 128, D), jnp.float32),
        scratch_types=[pltpu.VMEM((1, 64), jnp.int32),
                       pltpu.VMEM((1, 64), jnp.int32),
                       pltpu.VMEM((64, D), jnp.float32),
                       pltpu.SemaphoreType.DMA],
    )
    def k(x3_hbm, p0_hbm, p1_hbm, xg_hbm, p0v, p1v, rows_v, sem):
        wid = lax.axis_index("s") * 2 + lax.axis_index("c")
        pltpu.sync_copy(p0_hbm.at[pl.ds(wid, 1)], p0v)
        pltpu.sync_copy(p1_hbm.at[pl.ds(wid, 1)], p1v)
        pltpu.sync_copy(x3_hbm.at[pl.ds(wid * 64, 64)], rows_v)
        s0 = pltpu.async_copy(rows_v, xg_hbm.at[p0v.at[0]], sem)
        s1 = pltpu.async_copy(rows_v, xg_hbm.at[p1v.at[0]], sem)
        s0.wait()
        s1.wait()

    return k(x3, p0, p1)


def _k9_combine(yg, p0, p1, g0, g1, src2):
    @functools.partial(
        pl.kernel, mesh=_sc_mesh(),
        out_type=jax.ShapeDtypeStruct((S, D), jnp.float32),
        scratch_types=[pltpu.VMEM((4, 16), jnp.int32),
                       pltpu.VMEM((4, 16), jnp.int32),
                       pltpu.VMEM((4, 16), jnp.float32),
                       pltpu.VMEM((4, 16), jnp.float32),
                       pltpu.VMEM((16, D), jnp.float32),
                       pltpu.VMEM((16, D), jnp.float32),
                       pltpu.VMEM((16, D), jnp.float32),
                       pltpu.VMEM((16, D), jnp.float32),
                       pltpu.VMEM((16, D), jnp.float32),
                       pltpu.SemaphoreType.DMA,
                       pltpu.SemaphoreType.DMA],
    )
    def k(yg_hbm, p0_hbm, p1_hbm, g0_hbm, g1_hbm, src_hbm, out_hbm,
          p0v, p1v, g0v, g1v, r0a, r1a, r0b, r1b, sv, semA, semB):
        wid = lax.axis_index("s") * 2 + lax.axis_index("c")
        pltpu.sync_copy(p0_hbm.at[pl.ds(wid * 4, 4)], p0v)
        pltpu.sync_copy(p1_hbm.at[pl.ds(wid * 4, 4)], p1v)
        pltpu.sync_copy(g0_hbm.at[pl.ds(wid * 4, 4)], g0v)
        pltpu.sync_copy(g1_hbm.at[pl.ds(wid * 4, 4)], g1v)
        bufs = [(r0a, r1a, semA), (r0b, r1b, semB)]

        def issue(c):
            r0, r1, sm = bufs[c % 2]
            return (pltpu.async_copy(yg_hbm.at[p0v.at[c]], r0, sm),
                    pltpu.async_copy(yg_hbm.at[p1v.at[c]], r1, sm))

        pend = issue(0)
        for c in range(4):
            nxt = issue(c + 1) if c < 3 else None
            base = wid * 64 + c * 16
            pltpu.sync_copy(src_hbm.at[pl.ds(base, 16)], sv)
            g0row = g0v[c, :]
            g1row = g1v[c, :]
            ga = [jnp.full((16,), g0row[i], jnp.float32) for i in range(16)]
            gb = [jnp.full((16,), g1row[i], jnp.float32) for i in range(16)]
            pend[0].wait()
            pend[1].wait()
            r0v, r1v, _ = bufs[c % 2]

            def col(j, _2, r0v=r0v, r1v=r1v, ga=ga, gb=gb):
                sl = pl.ds(j * 16, 16)
                for i in range(16):
                    sv[i, sl] = sv[i, sl] + ga[i] * r0v[i, sl] + gb[i] * r1v[i, sl]
                return 0

            lax.fori_loop(0, D // 16, col, 0)
            pltpu.sync_copy(sv, out_hbm.at[pl.ds(base, 16)])
            pend = nxt

    return k(yg, p0, p1, g0, g1, src2)


def kernel(src, ln1_g, ln1_b, ln2_g, ln2_b, Wq, Wk, Wv, Wo, rel_bias,
           expert_sel, keys, values):
    src2d = src.reshape(S, D)
    q, k, v = _k1(src2d, ln1_g, ln1_b, Wq, Wk, Wv)
    t4 = _k2(rel_bias)
    att = _k3(q, k, v, t4)
    src2, x3, g0, g1, idx2 = _k45(att, Wo, src2d, ln2_g, ln2_b, expert_sel)
    pos0, pos1, eot = _k6_sort(idx2)
    xg = _k8_dispatch(x3, pos0.reshape(NW, 64), pos1.reshape(NW, 64))
    yg = _k7_group(xg, keys.astype(jnp.bfloat16), values.astype(jnp.bfloat16),
                   eot.reshape(NT))
    out = _k9_combine(yg,
                      pos0.reshape(NW * 4, 16),
                      pos1.reshape(NW * 4, 16),
                      g0.reshape(NW * 4, 16),
                      g1.reshape(NW * 4, 16),
                      src2)
    return out.reshape(1, S, D)


# speedup vs baseline: 1.6235x; 1.0695x over previous
import functools

import jax
import jax.numpy as jnp
import numpy as np
from jax import lax
from jax.experimental import pallas as pl
from jax.experimental.pallas import tpu as pltpu
from jax.experimental.pallas import tpu_sc as plsc

S, D, H, E, F = 2048, 768, 12, 64, 64
DH = D // H
NB = S // 128
ND = 2 * NB - 1
NP = 2 * S
NT = 96
NW = 32

_INTERPRET = False


def _pc(body, grid, in_specs, out_specs, out_shape, scratch_shapes=()):
    return pl.pallas_call(
        body,
        grid=grid,
        in_specs=in_specs,
        out_specs=out_specs,
        out_shape=out_shape,
        scratch_shapes=list(scratch_shapes),
        interpret=_INTERPRET,
    )


def _ln(x, g, b):
    m = jnp.mean(x, axis=-1, keepdims=True)
    v = jnp.mean((x - m) ** 2, axis=-1, keepdims=True)
    return (x - m) * jax.lax.rsqrt(v + 1e-5) * g + b


def _k1_body(src_ref, g_ref, b_ref, wq_ref, wk_ref, wv_ref, q_ref, k_ref, v_ref):
    x2 = _ln(src_ref[...], g_ref[...], b_ref[...]).astype(jnp.bfloat16)
    q = lax.dot(x2, wq_ref[...], preferred_element_type=jnp.float32) * 0.125
    k = lax.dot(x2, wk_ref[...], preferred_element_type=jnp.float32)
    v = lax.dot(x2, wv_ref[...], preferred_element_type=jnp.float32)
    qb, kb, vb = q.astype(jnp.bfloat16), k.astype(jnp.bfloat16), v.astype(jnp.bfloat16)
    ones = jnp.ones((128, 1), jnp.bfloat16)
    zeros = jnp.zeros((128, DH - 1), jnp.bfloat16)
    for h in range(H):
        sl = slice(h * DH, (h + 1) * DH)
        q_ref[h] = qb[:, sl]
        k_ref[h] = kb[:, sl]
        v_ref[h] = jnp.concatenate([vb[:, sl], ones, zeros], axis=1)


def _k1(src, ln1_g, ln1_b, wq, wk, wv):
    spec_w = pl.BlockSpec((D, D), lambda i: (0, 0))
    spec_v = pl.BlockSpec((1, D), lambda i: (0, 0))
    out_spec = pl.BlockSpec((H, 128, DH), lambda i: (0, i, 0))
    out_spec_v = pl.BlockSpec((H, 128, 2 * DH), lambda i: (0, i, 0))
    return _pc(
        _k1_body,
        grid=(NB,),
        in_specs=[pl.BlockSpec((128, D), lambda i: (i, 0)), spec_v, spec_v,
                  spec_w, spec_w, spec_w],
        out_specs=[out_spec, out_spec, out_spec_v],
        out_shape=[jax.ShapeDtypeStruct((H, S, DH), jnp.bfloat16),
                   jax.ShapeDtypeStruct((H, S, DH), jnp.bfloat16),
                   jax.ShapeDtypeStruct((H, S, 2 * DH), jnp.bfloat16)],
    )(src, ln1_g.reshape(1, D), ln1_b.reshape(1, D),
      wq.astype(jnp.bfloat16), wk.astype(jnp.bfloat16), wv.astype(jnp.bfloat16))


def _k3_body(q_ref, k_ref, v_ref, rb_ref, e_ref, o_ref, t_s):
    h = pl.program_id(0)
    i = pl.program_id(1)

    @pl.when((h == 0) & (i == 0))
    def _build_table():
        rb = rb_ref[...]
        sm = jnp.concatenate(
            [rb[:, 128 * d:128 * d + 256] for d in range(ND)], axis=0
        ).astype(jnp.bfloat16)
        for nc in range(4):
            sl = slice(nc * 4096, (nc + 1) * 4096)
            mm = lax.dot(sm, e_ref[:, sl], preferred_element_type=jnp.float32)
            t_s[:, 32 * nc:32 * (nc + 1), :] = (
                mm.astype(jnp.bfloat16).reshape(ND * H, 32, 128))

    q = q_ref[0]
    k = k_ref[0]
    s = lax.dot_general(q, k, (((1,), (1,)), ((), ())),
                        preferred_element_type=jnp.float32)
    patt = jnp.concatenate(
        [t_s[(j - i + (NB - 1)) * H + h].astype(jnp.float32)
         for j in range(NB)], axis=1)
    s = s + patt
    m = jnp.max(s, axis=1, keepdims=True)
    p = jnp.exp((s - m).astype(jnp.bfloat16))
    o2 = lax.dot(p, v_ref[0], preferred_element_type=jnp.float32)
    o = o2[:, :DH] * (1.0 / o2[:, DH:DH + 1])
    o_ref[0] = o.astype(jnp.bfloat16)


def _k3(q, k, v, rel_bias):
    rb = jnp.pad(rel_bias, ((0, 0), (0, 4096 - (2 * S - 1))))
    ab = np.arange(128 * 128)
    a, b = ab // 128, ab % 128
    c = np.arange(256)
    e_mat = (c[:, None] == (b - a + 127)[None, :]).astype(np.float32)
    e_mat = jnp.asarray(e_mat, dtype=jnp.bfloat16)
    return pl.pallas_call(
        _k3_body,
        grid=(H, NB),
        in_specs=[pl.BlockSpec((1, 128, DH), lambda h, i: (h, i, 0)),
                  pl.BlockSpec((1, S, DH), lambda h, i: (h, 0, 0)),
                  pl.BlockSpec((1, S, 2 * DH), lambda h, i: (h, 0, 0)),
                  pl.BlockSpec((H, 4096), lambda h, i: (0, 0)),
                  pl.BlockSpec((256, 128 * 128), lambda h, i: (0, 0))],
        out_specs=pl.BlockSpec((1, 128, DH), lambda h, i: (h, i, 0)),
        out_shape=jax.ShapeDtypeStruct((H, S, DH), jnp.bfloat16),
        scratch_shapes=[pltpu.VMEM((ND * H, 128, 128), jnp.bfloat16)],
        interpret=_INTERPRET,
    )(q, k, v, rb, e_mat)


def _k45_body(att_ref, wo_ref, src_ref, g_ref, b_ref, es_ref, lt_ref, sl_ref,
              src2_ref, x3_ref, g0_ref, g1_ref, e0_ref, e1_ref,
              r0_ref, r1_ref, sp_ref, eot_ref, cnt_s):
    i = pl.program_id(0)
    cat = jnp.concatenate([att_ref[h] for h in range(H)], axis=1)
    o = lax.dot(cat, wo_ref[...], preferred_element_type=jnp.float32)
    src2 = src_ref[...] + o
    src2_ref[...] = src2
    x3 = _ln(src2, g_ref[...], b_ref[...])
    x3_ref[...] = x3
    logits = lax.dot(x3, es_ref[...], preferred_element_type=jnp.float32)
    sel = jax.nn.sigmoid(logits)
    iota = lax.broadcasted_iota(jnp.int32, sel.shape, 1)
    m1 = jnp.max(sel, axis=1, keepdims=True)
    i1 = jnp.min(jnp.where(sel == m1, iota, E), axis=1, keepdims=True)
    masked = jnp.where(iota == i1, -1.0, sel)
    m2 = jnp.max(masked, axis=1, keepdims=True)
    i2 = jnp.min(jnp.where(masked == m2, iota, E), axis=1, keepdims=True)
    g0_ref[...] = m1
    g1_ref[...] = m2
    e0_ref[...] = i1
    e1_ref[...] = i2

    @pl.when(i == 0)
    def _init():
        cnt_s[...] = jnp.zeros((1, E), jnp.float32)

    cnt = cnt_s[...]
    oh0 = (i1 == iota).astype(jnp.float32)
    oh1 = (i2 == iota).astype(jnp.float32)
    ohs = oh0 + oh1
    cums = lax.dot(lt_ref[...], ohs.astype(jnp.bfloat16),
                   preferred_element_type=jnp.float32)
    r0_ref[...] = jnp.sum(oh0 * (cums + cnt), axis=1,
                          keepdims=True).astype(jnp.int32)
    r1_ref[...] = jnp.sum(oh1 * (cums + cnt + oh0), axis=1,
                          keepdims=True).astype(jnp.int32)
    newcnt = cnt + jnp.sum(ohs, axis=0, keepdims=True)
    cnt_s[...] = newcnt

    @pl.when(i == NB - 1)
    def _finalize():
        ntiles = jnp.ceil(newcnt * (1.0 / 128.0))
        tilestart = lax.dot(ntiles.astype(jnp.bfloat16), sl_ref[...],
                            preferred_element_type=jnp.float32)
        sp_ref[...] = (tilestart * 128.0).astype(jnp.int32)
        tio = lax.broadcasted_iota(jnp.int32, (NT, E), 0)
        eot_ref[...] = jnp.sum((tio >= tilestart.astype(jnp.int32))
                               .astype(jnp.int32), axis=1, keepdims=True) - 1


def _k45(att, wo, src, ln2_g, ln2_b, expert_sel):
    spec_v = pl.BlockSpec((1, D), lambda i: (0, 0))
    spec_1f = pl.BlockSpec((128, 1), lambda i: (i, 0))
    lt = jnp.asarray(np.tril(np.ones((128, 128), np.float32), -1),
                     dtype=jnp.bfloat16)
    sl = jnp.asarray(np.triu(np.ones((E, E), np.float32), 1),
                     dtype=jnp.bfloat16)
    return pl.pallas_call(
        _k45_body,
        grid=(NB,),
        in_specs=[pl.BlockSpec((H, 128, DH), lambda i: (0, i, 0)),
                  pl.BlockSpec((D, D), lambda i: (0, 0)),
                  pl.BlockSpec((128, D), lambda i: (i, 0)),
                  spec_v, spec_v,
                  pl.BlockSpec((D, E), lambda i: (0, 0)),
                  pl.BlockSpec((128, 128), lambda i: (0, 0)),
                  pl.BlockSpec((E, E), lambda i: (0, 0))],
        out_specs=[pl.BlockSpec((128, D), lambda i: (i, 0)),
                   pl.BlockSpec((128, D), lambda i: (i, 0)),
                   spec_1f, spec_1f, spec_1f, spec_1f, spec_1f, spec_1f,
                   pl.BlockSpec((1, E), lambda i: (0, 0)),
                   pl.BlockSpec((NT, 1), lambda i: (0, 0))],
        out_shape=[jax.ShapeDtypeStruct((S, D), jnp.float32),
                   jax.ShapeDtypeStruct((S, D), jnp.float32),
                   jax.ShapeDtypeStruct((S, 1), jnp.float32),
                   jax.ShapeDtypeStruct((S, 1), jnp.float32),
                   jax.ShapeDtypeStruct((S, 1), jnp.int32),
                   jax.ShapeDtypeStruct((S, 1), jnp.int32),
                   jax.ShapeDtypeStruct((S, 1), jnp.int32),
                   jax.ShapeDtypeStruct((S, 1), jnp.int32),
                   jax.ShapeDtypeStruct((1, E), jnp.int32),
                   jax.ShapeDtypeStruct((NT, 1), jnp.int32)],
        scratch_shapes=[pltpu.VMEM((1, E), jnp.float32)],
        interpret=_INTERPRET,
    )(att, wo.astype(jnp.bfloat16), src, ln2_g.reshape(1, D),
      ln2_b.reshape(1, D), expert_sel, lt, sl)


def _k7_body(eot_sref, xg_ref, k_ref, v_ref, yg_ref):
    x = xg_ref[...].astype(jnp.bfloat16)
    hid = jax.nn.relu(lax.dot(x, k_ref[0], preferred_element_type=jnp.float32))
    yg_ref[...] = lax.dot(hid.astype(jnp.bfloat16), v_ref[0],
                          preferred_element_type=jnp.float32)


def _k7_group(xg, keys_bf, values_bf, eot_flat):
    grid_spec = pltpu.PrefetchScalarGridSpec(
        num_scalar_prefetch=1,
        grid=(NT,),
        in_specs=[pl.BlockSpec((128, D), lambda t, e: (t, 0)),
                  pl.BlockSpec((1, D, F), lambda t, e: (e[t], 0, 0)),
                  pl.BlockSpec((1, F, D), lambda t, e: (e[t], 0, 0))],
        out_specs=pl.BlockSpec((128, D), lambda t, e: (t, 0)),
    )
    return pl.pallas_call(
        _k7_body,
        grid_spec=grid_spec,
        out_shape=jax.ShapeDtypeStruct((NT * 128, D), jnp.float32),
        interpret=_INTERPRET,
    )(eot_flat, xg, keys_bf, values_bf)


def _sc_mesh():
    return plsc.VectorSubcoreMesh(core_axis_name="c", subcore_axis_name="s",
                                  num_cores=2)


def _k8_dispatch(x3, r0, r1, e0, e1, sp):
    @functools.partial(
        pl.kernel, mesh=_sc_mesh(),
        out_type=[jax.ShapeDtypeStruct((NT * 128, D), jnp.float32),
                  jax.ShapeDtypeStruct((NW, 64), jnp.int32),
                  jax.ShapeDtypeStruct((NW, 64), jnp.int32)],
        scratch_types=[pltpu.VMEM((1, 64), jnp.int32),
                       pltpu.VMEM((1, 64), jnp.int32),
                       pltpu.VMEM((1, 64), jnp.int32),
                       pltpu.VMEM((1, 64), jnp.int32),
                       pltpu.VMEM((1, 64), jnp.int32),
                       pltpu.VMEM((1, 64), jnp.int32),
                       pltpu.VMEM((1, 64), jnp.int32),
                       pltpu.VMEM((64, D), jnp.float32),
                       pltpu.SemaphoreType.DMA],
    )
    def k(x3_hbm, r0_hbm, r1_hbm, e0_hbm, e1_hbm, sp_hbm,
          xg_hbm, p0f_hbm, p1f_hbm,
          spv, r0v, r1v, e0v, e1v, p0v, p1v, rows_v, sem):
        wid = lax.axis_index("s") * 2 + lax.axis_index("c")
        pltpu.sync_copy(sp_hbm.at[pl.ds(0, 1)], spv)
        pltpu.sync_copy(r0_hbm.at[pl.ds(wid, 1)], r0v)
        pltpu.sync_copy(r1_hbm.at[pl.ds(wid, 1)], r1v)
        pltpu.sync_copy(e0_hbm.at[pl.ds(wid, 1)], e0v)
        pltpu.sync_copy(e1_hbm.at[pl.ds(wid, 1)], e1v)
        sp_chunks = [spv[0, pl.ds(k * 16, 16)] for k in range(4)]
        sp_scalar = [sp_chunks[j // 16][j % 16] for j in range(E)]
        for c in range(4):
            slc = pl.ds(c * 16, 16)
            e0c = e0v[0, slc]
            e1c = e1v[0, slc]
            acc0 = jnp.zeros((16,), jnp.int32)
            acc1 = jnp.zeros((16,), jnp.int32)
            for j in range(E):
                acc0 = jnp.where(e0c == j, sp_scalar[j], acc0)
                acc1 = jnp.where(e1c == j, sp_scalar[j], acc1)
            p0v[0, slc] = r0v[0, slc] + acc0
            p1v[0, slc] = r1v[0, slc] + acc1
        pltpu.sync_copy(x3_hbm.at[pl.ds(wid * 64, 64)], rows_v)
        s0 = pltpu.async_copy(rows_v, xg_hbm.at[p0v.at[0]], sem)
        s1 = pltpu.async_copy(rows_v, xg_hbm.at[p1v.at[0]], sem)
        pltpu.sync_copy(p0v, p0f_hbm.at[pl.ds(wid, 1)])
        pltpu.sync_copy(p1v, p1f_hbm.at[pl.ds(wid, 1)])
        s0.wait()
        s1.wait()

    return k(x3, r0, r1, e0, e1, sp)


def _k9_combine(yg, p0, p1, g0, g1, src2):
    @functools.partial(
        pl.kernel, mesh=_sc_mesh(),
        out_type=jax.ShapeDtypeStruct((S, D), jnp.float32),
        scratch_types=[pltpu.VMEM((4, 16), jnp.int32),
                       pltpu.VMEM((4, 16), jnp.int32),
                       pltpu.VMEM((4, 16), jnp.float32),
                       pltpu.VMEM((4, 16), jnp.float32),
                       pltpu.VMEM((16, D), jnp.float32),
                       pltpu.VMEM((16, D), jnp.float32),
                       pltpu.VMEM((16, D), jnp.float32),
                       pltpu.VMEM((16, D), jnp.float32),
                       pltpu.VMEM((16, D), jnp.float32),
                       pltpu.SemaphoreType.DMA,
                       pltpu.SemaphoreType.DMA],
    )
    def k(yg_hbm, p0_hbm, p1_hbm, g0_hbm, g1_hbm, src_hbm, out_hbm,
          p0v, p1v, g0v, g1v, r0a, r1a, r0b, r1b, sv, semA, semB):
        wid = lax.axis_index("s") * 2 + lax.axis_index("c")
        pltpu.sync_copy(p0_hbm.at[pl.ds(wid * 4, 4)], p0v)
        pltpu.sync_copy(p1_hbm.at[pl.ds(wid * 4, 4)], p1v)
        pltpu.sync_copy(g0_hbm.at[pl.ds(wid * 4, 4)], g0v)
        pltpu.sync_copy(g1_hbm.at[pl.ds(wid * 4, 4)], g1v)
        bufs = [(r0a, r1a, semA), (r0b, r1b, semB)]

        def issue(c):
            r0, r1, sm = bufs[c % 2]
            return (pltpu.async_copy(yg_hbm.at[p0v.at[c]], r0, sm),
                    pltpu.async_copy(yg_hbm.at[p1v.at[c]], r1, sm))

        pend = issue(0)
        for c in range(4):
            nxt = issue(c + 1) if c < 3 else None
            base = wid * 64 + c * 16
            pltpu.sync_copy(src_hbm.at[pl.ds(base, 16)], sv)
            g0row = g0v[c, :]
            g1row = g1v[c, :]
            ga = [jnp.full((16,), g0row[i], jnp.float32) for i in range(16)]
            gb = [jnp.full((16,), g1row[i], jnp.float32) for i in range(16)]
            pend[0].wait()
            pend[1].wait()
            r0v, r1v, _ = bufs[c % 2]

            def col(j, _2, r0v=r0v, r1v=r1v, ga=ga, gb=gb):
                sl = pl.ds(j * 16, 16)
                for i in range(16):
                    sv[i, sl] = sv[i, sl] + ga[i] * r0v[i, sl] + gb[i] * r1v[i, sl]
                return 0

            lax.fori_loop(0, D // 16, col, 0)
            pltpu.sync_copy(sv, out_hbm.at[pl.ds(base, 16)])
            pend = nxt

    return k(yg, p0, p1, g0, g1, src2)


def kernel(src, ln1_g, ln1_b, ln2_g, ln2_b, Wq, Wk, Wv, Wo, rel_bias,
           expert_sel, keys, values):
    src2d = src.reshape(S, D)
    q, k, v = _k1(src2d, ln1_g, ln1_b, Wq, Wk, Wv)
    att = _k3(q, k, v, rel_bias)
    (src2, x3, g0, g1, e0, e1, r0, r1, sp, eot) = _k45(
        att, Wo, src2d, ln2_g, ln2_b, expert_sel)
    xg, p0f, p1f = _k8_dispatch(x3, r0.reshape(NW, 64), r1.reshape(NW, 64),
                                e0.reshape(NW, 64), e1.reshape(NW, 64), sp)
    yg = _k7_group(xg, keys.astype(jnp.bfloat16), values.astype(jnp.bfloat16),
                   eot.reshape(NT))
    out = _k9_combine(yg,
                      p0f.reshape(NW * 4, 16),
                      p1f.reshape(NW * 4, 16),
                      g0.reshape(NW * 4, 16),
                      g1.reshape(NW * 4, 16),
                      src2)
    return out.reshape(1, S, D)


# speedup vs baseline: 1.6356x; 1.0074x over previous
import functools

import jax
import jax.numpy as jnp
import numpy as np
from jax import lax
from jax.experimental import pallas as pl
from jax.experimental.pallas import tpu as pltpu
from jax.experimental.pallas import tpu_sc as plsc

S, D, H, E, F = 2048, 768, 12, 64, 64
DH = D // H
NB = S // 128
ND = 2 * NB - 1
NP = 2 * S
NT = 96
NW = 32

_INTERPRET = False


def _pc(body, grid, in_specs, out_specs, out_shape, scratch_shapes=()):
    return pl.pallas_call(
        body,
        grid=grid,
        in_specs=in_specs,
        out_specs=out_specs,
        out_shape=out_shape,
        scratch_shapes=list(scratch_shapes),
        interpret=_INTERPRET,
    )


def _ln(x, g, b):
    m = jnp.mean(x, axis=-1, keepdims=True)
    v = jnp.mean((x - m) ** 2, axis=-1, keepdims=True)
    return (x - m) * jax.lax.rsqrt(v + 1e-5) * g + b


def _k1_body(src_ref, g_ref, b_ref, wq_ref, wk_ref, wv_ref, q_ref, k_ref, v_ref):
    x2 = _ln(src_ref[...], g_ref[...], b_ref[...]).astype(jnp.bfloat16)
    q = lax.dot(x2, wq_ref[...], preferred_element_type=jnp.float32) * 0.125
    k = lax.dot(x2, wk_ref[...], preferred_element_type=jnp.float32)
    v = lax.dot(x2, wv_ref[...], preferred_element_type=jnp.float32)
    qb, kb, vb = q.astype(jnp.bfloat16), k.astype(jnp.bfloat16), v.astype(jnp.bfloat16)
    ones = jnp.ones((128, 1), jnp.bfloat16)
    zeros = jnp.zeros((128, DH - 1), jnp.bfloat16)
    for h in range(H):
        sl = slice(h * DH, (h + 1) * DH)
        q_ref[h] = qb[:, sl]
        k_ref[h] = kb[:, sl]
        v_ref[h] = jnp.concatenate([vb[:, sl], ones, zeros], axis=1)


def _k1(src, ln1_g, ln1_b, wq, wk, wv):
    spec_w = pl.BlockSpec((D, D), lambda i: (0, 0))
    spec_v = pl.BlockSpec((1, D), lambda i: (0, 0))
    out_spec = pl.BlockSpec((H, 128, DH), lambda i: (0, i, 0))
    out_spec_v = pl.BlockSpec((H, 128, 2 * DH), lambda i: (0, i, 0))
    return _pc(
        _k1_body,
        grid=(NB,),
        in_specs=[pl.BlockSpec((128, D), lambda i: (i, 0)), spec_v, spec_v,
                  spec_w, spec_w, spec_w],
        out_specs=[out_spec, out_spec, out_spec_v],
        out_shape=[jax.ShapeDtypeStruct((H, S, DH), jnp.bfloat16),
                   jax.ShapeDtypeStruct((H, S, DH), jnp.bfloat16),
                   jax.ShapeDtypeStruct((H, S, 2 * DH), jnp.bfloat16)],
    )(src, ln1_g.reshape(1, D), ln1_b.reshape(1, D),
      wq.astype(jnp.bfloat16), wk.astype(jnp.bfloat16), wv.astype(jnp.bfloat16))


def _k3_body(q_ref, k_ref, v_ref, rb_ref, e_ref, o_ref, t_s):
    h = pl.program_id(0)
    i = pl.program_id(1)

    @pl.when((h == 0) & (i == 0))
    def _build_table():
        rb = rb_ref[...]
        sm = jnp.concatenate(
            [rb[:, 128 * d:128 * d + 256] for d in range(ND)], axis=0
        ).astype(jnp.bfloat16)
        for nc in range(4):
            sl = slice(nc * 4096, (nc + 1) * 4096)
            mm = lax.dot(sm, e_ref[:, sl], preferred_element_type=jnp.float32)
            t_s[:, 32 * nc:32 * (nc + 1), :] = (
                mm.astype(jnp.bfloat16).reshape(ND * H, 32, 128))

    q = q_ref[0]
    k = k_ref[0]
    s = lax.dot_general(q, k, (((1,), (1,)), ((), ())),
                        preferred_element_type=jnp.float32)
    patt = jnp.concatenate(
        [t_s[(j - i + (NB - 1)) * H + h] for j in range(NB)], axis=1)
    m = jnp.max(s, axis=1, keepdims=True) + 0.5
    p = jnp.exp((s - m).astype(jnp.bfloat16) + patt)
    o2 = lax.dot(p, v_ref[0], preferred_element_type=jnp.float32)
    o = o2[:, :DH] * (1.0 / o2[:, DH:DH + 1])
    o_ref[0] = o.astype(jnp.bfloat16)


def _k3(q, k, v, rel_bias):
    rb = jnp.pad(rel_bias, ((0, 0), (0, 4096 - (2 * S - 1))))
    ab = np.arange(128 * 128)
    a, b = ab // 128, ab % 128
    c = np.arange(256)
    e_mat = (c[:, None] == (b - a + 127)[None, :]).astype(np.float32)
    e_mat = jnp.asarray(e_mat, dtype=jnp.bfloat16)
    return pl.pallas_call(
        _k3_body,
        grid=(H, NB),
        in_specs=[pl.BlockSpec((1, 128, DH), lambda h, i: (h, i, 0)),
                  pl.BlockSpec((1, S, DH), lambda h, i: (h, 0, 0)),
                  pl.BlockSpec((1, S, 2 * DH), lambda h, i: (h, 0, 0)),
                  pl.BlockSpec((H, 4096), lambda h, i: (0, 0)),
                  pl.BlockSpec((256, 128 * 128), lambda h, i: (0, 0))],
        out_specs=pl.BlockSpec((1, 128, DH), lambda h, i: (h, i, 0)),
        out_shape=jax.ShapeDtypeStruct((H, S, DH), jnp.bfloat16),
        scratch_shapes=[pltpu.VMEM((ND * H, 128, 128), jnp.bfloat16)],
        interpret=_INTERPRET,
    )(q, k, v, rb, e_mat)


def _k45_body(att_ref, wo_ref, src_ref, g_ref, b_ref, es_ref, lt_ref, sl_ref,
              src2_ref, x3_ref, g0_ref, g1_ref, e0_ref, e1_ref,
              r0_ref, r1_ref, sp_ref, eot_ref, cnt_s):
    i = pl.program_id(0)
    cat = jnp.concatenate([att_ref[h] for h in range(H)], axis=1)
    o = lax.dot(cat, wo_ref[...], preferred_element_type=jnp.float32)
    src2 = src_ref[...] + o
    src2_ref[...] = src2
    x3 = _ln(src2, g_ref[...], b_ref[...])
    x3_ref[...] = x3
    logits = lax.dot(x3, es_ref[...], preferred_element_type=jnp.float32)
    sel = jax.nn.sigmoid(logits)
    iota = lax.broadcasted_iota(jnp.int32, sel.shape, 1)
    m1 = jnp.max(sel, axis=1, keepdims=True)
    i1 = jnp.min(jnp.where(sel == m1, iota, E), axis=1, keepdims=True)
    masked = jnp.where(iota == i1, -1.0, sel)
    m2 = jnp.max(masked, axis=1, keepdims=True)
    i2 = jnp.min(jnp.where(masked == m2, iota, E), axis=1, keepdims=True)
    g0_ref[...] = m1
    g1_ref[...] = m2
    e0_ref[...] = i1
    e1_ref[...] = i2

    @pl.when(i == 0)
    def _init():
        cnt_s[...] = jnp.zeros((1, E), jnp.float32)

    cnt = cnt_s[...]
    oh0 = (i1 == iota).astype(jnp.float32)
    oh1 = (i2 == iota).astype(jnp.float32)
    ohs = oh0 + oh1
    cums = lax.dot(lt_ref[...], ohs.astype(jnp.bfloat16),
                   preferred_element_type=jnp.float32)
    r0_ref[...] = jnp.sum(oh0 * (cums + cnt), axis=1,
                          keepdims=True).astype(jnp.int32)
    r1_ref[...] = jnp.sum(oh1 * (cums + cnt + oh0), axis=1,
                          keepdims=True).astype(jnp.int32)
    newcnt = cnt + jnp.sum(ohs, axis=0, keepdims=True)
    cnt_s[...] = newcnt

    @pl.when(i == NB - 1)
    def _finalize():
        ntiles = jnp.ceil(newcnt * (1.0 / 128.0))
        tilestart = lax.dot(ntiles.astype(jnp.bfloat16), sl_ref[...],
                            preferred_element_type=jnp.float32)
        sp_ref[...] = (tilestart * 128.0).astype(jnp.int32)
        tio = lax.broadcasted_iota(jnp.int32, (NT, E), 0)
        eot_ref[...] = jnp.sum((tio >= tilestart.astype(jnp.int32))
                               .astype(jnp.int32), axis=1, keepdims=True) - 1


def _k45(att, wo, src, ln2_g, ln2_b, expert_sel):
    spec_v = pl.BlockSpec((1, D), lambda i: (0, 0))
    spec_1f = pl.BlockSpec((128, 1), lambda i: (i, 0))
    lt = jnp.asarray(np.tril(np.ones((128, 128), np.float32), -1),
                     dtype=jnp.bfloat16)
    sl = jnp.asarray(np.triu(np.ones((E, E), np.float32), 1),
                     dtype=jnp.bfloat16)
    return pl.pallas_call(
        _k45_body,
        grid=(NB,),
        in_specs=[pl.BlockSpec((H, 128, DH), lambda i: (0, i, 0)),
                  pl.BlockSpec((D, D), lambda i: (0, 0)),
                  pl.BlockSpec((128, D), lambda i: (i, 0)),
                  spec_v, spec_v,
                  pl.BlockSpec((D, E), lambda i: (0, 0)),
                  pl.BlockSpec((128, 128), lambda i: (0, 0)),
                  pl.BlockSpec((E, E), lambda i: (0, 0))],
        out_specs=[pl.BlockSpec((128, D), lambda i: (i, 0)),
                   pl.BlockSpec((128, D), lambda i: (i, 0)),
                   spec_1f, spec_1f, spec_1f, spec_1f, spec_1f, spec_1f,
                   pl.BlockSpec((1, E), lambda i: (0, 0)),
                   pl.BlockSpec((NT, 1), lambda i: (0, 0))],
        out_shape=[jax.ShapeDtypeStruct((S, D), jnp.float32),
                   jax.ShapeDtypeStruct((S, D), jnp.float32),
                   jax.ShapeDtypeStruct((S, 1), jnp.float32),
                   jax.ShapeDtypeStruct((S, 1), jnp.float32),
                   jax.ShapeDtypeStruct((S, 1), jnp.int32),
                   jax.ShapeDtypeStruct((S, 1), jnp.int32),
                   jax.ShapeDtypeStruct((S, 1), jnp.int32),
                   jax.ShapeDtypeStruct((S, 1), jnp.int32),
                   jax.ShapeDtypeStruct((1, E), jnp.int32),
                   jax.ShapeDtypeStruct((NT, 1), jnp.int32)],
        scratch_shapes=[pltpu.VMEM((1, E), jnp.float32)],
        interpret=_INTERPRET,
    )(att, wo.astype(jnp.bfloat16), src, ln2_g.reshape(1, D),
      ln2_b.reshape(1, D), expert_sel, lt, sl)


def _k7_body(eot_sref, xg_ref, k_ref, v_ref, yg_ref):
    x = xg_ref[...].astype(jnp.bfloat16)
    hid = jax.nn.relu(lax.dot(x, k_ref[0], preferred_element_type=jnp.float32))
    yg_ref[...] = lax.dot(hid.astype(jnp.bfloat16), v_ref[0],
                          preferred_element_type=jnp.float32)


def _k7_group(xg, keys_bf, values_bf, eot_flat):
    grid_spec = pltpu.PrefetchScalarGridSpec(
        num_scalar_prefetch=1,
        grid=(NT,),
        in_specs=[pl.BlockSpec((128, D), lambda t, e: (t, 0)),
                  pl.BlockSpec((1, D, F), lambda t, e: (e[t], 0, 0)),
                  pl.BlockSpec((1, F, D), lambda t, e: (e[t], 0, 0))],
        out_specs=pl.BlockSpec((128, D), lambda t, e: (t, 0)),
    )
    return pl.pallas_call(
        _k7_body,
        grid_spec=grid_spec,
        out_shape=jax.ShapeDtypeStruct((NT * 128, D), jnp.float32),
        interpret=_INTERPRET,
    )(eot_flat, xg, keys_bf, values_bf)


def _sc_mesh():
    return plsc.VectorSubcoreMesh(core_axis_name="c", subcore_axis_name="s",
                                  num_cores=2)


def _k8_dispatch(x3, r0, r1, e0, e1, sp):
    @functools.partial(
        pl.kernel, mesh=_sc_mesh(),
        out_type=[jax.ShapeDtypeStruct((NT * 128, D), jnp.float32),
                  jax.ShapeDtypeStruct((NW, 64), jnp.int32),
                  jax.ShapeDtypeStruct((NW, 64), jnp.int32)],
        scratch_types=[pltpu.VMEM((1, 64), jnp.int32),
                       pltpu.VMEM((1, 64), jnp.int32),
                       pltpu.VMEM((1, 64), jnp.int32),
                       pltpu.VMEM((1, 64), jnp.int32),
                       pltpu.VMEM((1, 64), jnp.int32),
                       pltpu.VMEM((1, 64), jnp.int32),
                       pltpu.VMEM((1, 64), jnp.int32),
                       pltpu.VMEM((64, D), jnp.float32),
                       pltpu.SemaphoreType.DMA],
    )
    def k(x3_hbm, r0_hbm, r1_hbm, e0_hbm, e1_hbm, sp_hbm,
          xg_hbm, p0f_hbm, p1f_hbm,
          spv, r0v, r1v, e0v, e1v, p0v, p1v, rows_v, sem):
        wid = lax.axis_index("s") * 2 + lax.axis_index("c")
        pltpu.sync_copy(sp_hbm.at[pl.ds(0, 1)], spv)
        pltpu.sync_copy(r0_hbm.at[pl.ds(wid, 1)], r0v)
        pltpu.sync_copy(r1_hbm.at[pl.ds(wid, 1)], r1v)
        pltpu.sync_copy(e0_hbm.at[pl.ds(wid, 1)], e0v)
        pltpu.sync_copy(e1_hbm.at[pl.ds(wid, 1)], e1v)
        sp_chunks = [spv[0, pl.ds(k * 16, 16)] for k in range(4)]
        sp_scalar = [sp_chunks[j // 16][j % 16] for j in range(E)]
        for c in range(4):
            slc = pl.ds(c * 16, 16)
            e0c = e0v[0, slc]
            e1c = e1v[0, slc]
            acc0 = jnp.zeros((16,), jnp.int32)
            acc1 = jnp.zeros((16,), jnp.int32)
            for j in range(E):
                acc0 = jnp.where(e0c == j, sp_scalar[j], acc0)
                acc1 = jnp.where(e1c == j, sp_scalar[j], acc1)
            p0v[0, slc] = r0v[0, slc] + acc0
            p1v[0, slc] = r1v[0, slc] + acc1
        pltpu.sync_copy(x3_hbm.at[pl.ds(wid * 64, 64)], rows_v)
        s0 = pltpu.async_copy(rows_v, xg_hbm.at[p0v.at[0]], sem)
        s1 = pltpu.async_copy(rows_v, xg_hbm.at[p1v.at[0]], sem)
        pltpu.sync_copy(p0v, p0f_hbm.at[pl.ds(wid, 1)])
        pltpu.sync_copy(p1v, p1f_hbm.at[pl.ds(wid, 1)])
        s0.wait()
        s1.wait()

    return k(x3, r0, r1, e0, e1, sp)


def _k9_combine(yg, p0, p1, g0, g1, src2):
    @functools.partial(
        pl.kernel, mesh=_sc_mesh(),
        out_type=jax.ShapeDtypeStruct((S, D), jnp.float32),
        scratch_types=[pltpu.VMEM((4, 16), jnp.int32),
                       pltpu.VMEM((4, 16), jnp.int32),
                       pltpu.VMEM((4, 16), jnp.float32),
                       pltpu.VMEM((4, 16), jnp.float32),
                       pltpu.VMEM((16, D), jnp.float32),
                       pltpu.VMEM((16, D), jnp.float32),
                       pltpu.VMEM((16, D), jnp.float32),
                       pltpu.VMEM((16, D), jnp.float32),
                       pltpu.VMEM((16, D), jnp.float32),
                       pltpu.SemaphoreType.DMA,
                       pltpu.SemaphoreType.DMA],
    )
    def k(yg_hbm, p0_hbm, p1_hbm, g0_hbm, g1_hbm, src_hbm, out_hbm,
          p0v, p1v, g0v, g1v, r0a, r1a, r0b, r1b, sv, semA, semB):
        wid = lax.axis_index("s") * 2 + lax.axis_index("c")
        pltpu.sync_copy(p0_hbm.at[pl.ds(wid * 4, 4)], p0v)
        pltpu.sync_copy(p1_hbm.at[pl.ds(wid * 4, 4)], p1v)
        pltpu.sync_copy(g0_hbm.at[pl.ds(wid * 4, 4)], g0v)
        pltpu.sync_copy(g1_hbm.at[pl.ds(wid * 4, 4)], g1v)
        bufs = [(r0a, r1a, semA), (r0b, r1b, semB)]

        def issue(c):
            r0, r1, sm = bufs[c % 2]
            return (pltpu.async_copy(yg_hbm.at[p0v.at[c]], r0, sm),
                    pltpu.async_copy(yg_hbm.at[p1v.at[c]], r1, sm))

        pend = issue(0)
        for c in range(4):
            nxt = issue(c + 1) if c < 3 else None
            base = wid * 64 + c * 16
            pltpu.sync_copy(src_hbm.at[pl.ds(base, 16)], sv)
            g0row = g0v[c, :]
            g1row = g1v[c, :]
            ga = [jnp.full((16,), g0row[i], jnp.float32) for i in range(16)]
            gb = [jnp.full((16,), g1row[i], jnp.float32) for i in range(16)]
            pend[0].wait()
            pend[1].wait()
            r0v, r1v, _ = bufs[c % 2]

            def col(j, _2, r0v=r0v, r1v=r1v, ga=ga, gb=gb):
                sl = pl.ds(j * 16, 16)
                for i in range(16):
                    sv[i, sl] = sv[i, sl] + ga[i] * r0v[i, sl] + gb[i] * r1v[i, sl]
                return 0

            lax.fori_loop(0, D // 16, col, 0)
            pltpu.sync_copy(sv, out_hbm.at[pl.ds(base, 16)])
            pend = nxt

    return k(yg, p0, p1, g0, g1, src2)


def kernel(src, ln1_g, ln1_b, ln2_g, ln2_b, Wq, Wk, Wv, Wo, rel_bias,
           expert_sel, keys, values):
    src2d = src.reshape(S, D)
    q, k, v = _k1(src2d, ln1_g, ln1_b, Wq, Wk, Wv)
    att = _k3(q, k, v, rel_bias)
    (src2, x3, g0, g1, e0, e1, r0, r1, sp, eot) = _k45(
        att, Wo, src2d, ln2_g, ln2_b, expert_sel)
    xg, p0f, p1f = _k8_dispatch(x3, r0.reshape(NW, 64), r1.reshape(NW, 64),
                                e0.reshape(NW, 64), e1.reshape(NW, 64), sp)
    yg = _k7_group(xg, keys.astype(jnp.bfloat16), values.astype(jnp.bfloat16),
                   eot.reshape(NT))
    out = _k9_combine(yg,
                      p0f.reshape(NW * 4, 16),
                      p1f.reshape(NW * 4, 16),
                      g0.reshape(NW * 4, 16),
                      g1.reshape(NW * 4, 16),
                      src2)
    return out.reshape(1, S, D)


# speedup vs baseline: 1.6815x; 1.0281x over previous
import functools

import jax
import jax.numpy as jnp
import numpy as np
from jax import lax
from jax.experimental import pallas as pl
from jax.experimental.pallas import tpu as pltpu
from jax.experimental.pallas import tpu_sc as plsc

S, D, H, E, F = 2048, 768, 12, 64, 64
DH = D // H
NB = S // 128
ND = 2 * NB - 1
NP = 2 * S
NT = 96
NW = 32

_INTERPRET = False


def _pc(body, grid, in_specs, out_specs, out_shape, scratch_shapes=()):
    return pl.pallas_call(
        body,
        grid=grid,
        in_specs=in_specs,
        out_specs=out_specs,
        out_shape=out_shape,
        scratch_shapes=list(scratch_shapes),
        interpret=_INTERPRET,
    )


def _ln(x, g, b):
    m = jnp.mean(x, axis=-1, keepdims=True)
    v = jnp.mean((x - m) ** 2, axis=-1, keepdims=True)
    return (x - m) * jax.lax.rsqrt(v + 1e-5) * g + b


def _mega_body(src_ref, ln1g_ref, ln1b_ref, wq_ref, wk_ref, wv_ref,
               rb_ref, wo_ref, ln2g_ref, ln2b_ref, es_ref,
               lt_ref, sl_ref,
               src2_ref, x3_ref, g0_ref, g1_ref, e0_ref, e1_ref,
               r0_ref, r1_ref, sp_ref, eot_ref,
               t_s, q_s, k_s, v_s, att_s, cnt_s):
    i = pl.program_id(0)
    h = pl.program_id(1)

    @pl.when((i == 0) & (h == 0))
    def _prep():
        rb = rb_ref[...]
        sm = jnp.concatenate(
            [rb[:, 128 * d:128 * d + 256] for d in range(ND)], axis=0
        ).astype(jnp.bfloat16)
        for nc in range(4):
            x = lax.broadcasted_iota(jnp.int32, (256, 4096), 1) + 4096 * nc
            cidx = lax.broadcasted_iota(jnp.int32, (256, 4096), 0)
            e_chunk = (cidx == ((x & 127) - (x >> 7) + 127)).astype(jnp.bfloat16)
            mm = lax.dot(sm, e_chunk, preferred_element_type=jnp.float32)
            t_s[:, 32 * nc:32 * (nc + 1), :] = (
                mm.astype(jnp.bfloat16).reshape(ND * H, 32, 128))
        ones = jnp.ones((128, 1), jnp.bfloat16)
        zeros = jnp.zeros((128, DH - 1), jnp.bfloat16)
        for b in range(NB):
            x2 = _ln(src_ref[pl.ds(128 * b, 128), :], ln1g_ref[...],
                     ln1b_ref[...]).astype(jnp.bfloat16)
            qb = (lax.dot(x2, wq_ref[...], preferred_element_type=jnp.float32)
                  * 0.125).astype(jnp.bfloat16)
            kb = lax.dot(x2, wk_ref[...],
                         preferred_element_type=jnp.float32).astype(jnp.bfloat16)
            vb = lax.dot(x2, wv_ref[...],
                         preferred_element_type=jnp.float32).astype(jnp.bfloat16)
            for hh in range(H):
                cs = slice(hh * DH, (hh + 1) * DH)
                q_s[hh, pl.ds(128 * b, 128), :] = qb[:, cs]
                k_s[hh, pl.ds(128 * b, 128), :] = kb[:, cs]
                v_s[hh, pl.ds(128 * b, 128), :] = jnp.concatenate(
                    [vb[:, cs], ones, zeros], axis=1)

    q = q_s[h, pl.ds(128 * i, 128), :]
    k = k_s[h]
    s = lax.dot_general(q, k, (((1,), (1,)), ((), ())),
                        preferred_element_type=jnp.float32)
    patt = jnp.concatenate(
        [t_s[(j - i + (NB - 1)) * H + h] for j in range(NB)], axis=1)
    m = jnp.max(s, axis=1, keepdims=True) + 0.5
    p = jnp.exp((s - m).astype(jnp.bfloat16) + patt)
    o2 = lax.dot(p, v_s[h], preferred_element_type=jnp.float32)
    att_s[h] = (o2[:, :DH] * (1.0 / o2[:, DH:DH + 1])).astype(jnp.bfloat16)

    @pl.when(h == H - 1)
    def _tail():
        cat = jnp.concatenate([att_s[hh] for hh in range(H)], axis=1)
        o = lax.dot(cat, wo_ref[...], preferred_element_type=jnp.float32)
        src2 = src_ref[pl.ds(128 * i, 128), :] + o
        src2_ref[...] = src2
        x3 = _ln(src2, ln2g_ref[...], ln2b_ref[...])
        x3_ref[...] = x3
        logits = lax.dot(x3, es_ref[...], preferred_element_type=jnp.float32)
        sel = jax.nn.sigmoid(logits)
        iota = lax.broadcasted_iota(jnp.int32, sel.shape, 1)
        m1 = jnp.max(sel, axis=1, keepdims=True)
        i1 = jnp.min(jnp.where(sel == m1, iota, E), axis=1, keepdims=True)
        masked = jnp.where(iota == i1, -1.0, sel)
        m2 = jnp.max(masked, axis=1, keepdims=True)
        i2 = jnp.min(jnp.where(masked == m2, iota, E), axis=1, keepdims=True)
        g0_ref[...] = m1
        g1_ref[...] = m2
        e0_ref[...] = i1
        e1_ref[...] = i2
        cnt = jnp.where(i == 0, jnp.zeros((1, E), jnp.float32), cnt_s[...])
        oh0 = (i1 == iota).astype(jnp.float32)
        oh1 = (i2 == iota).astype(jnp.float32)
        ohs = oh0 + oh1
        cums = lax.dot(lt_ref[...], ohs.astype(jnp.bfloat16),
                       preferred_element_type=jnp.float32)
        r0_ref[...] = jnp.sum(oh0 * (cums + cnt), axis=1,
                              keepdims=True).astype(jnp.int32)
        r1_ref[...] = jnp.sum(oh1 * (cums + cnt + oh0), axis=1,
                              keepdims=True).astype(jnp.int32)
        newcnt = cnt + jnp.sum(ohs, axis=0, keepdims=True)
        cnt_s[...] = newcnt

        @pl.when(i == NB - 1)
        def _finalize():
            ntiles = jnp.ceil(newcnt * (1.0 / 128.0))
            tilestart = lax.dot(ntiles.astype(jnp.bfloat16), sl_ref[...],
                                preferred_element_type=jnp.float32)
            sp_ref[...] = (tilestart * 128.0).astype(jnp.int32)
            tio = lax.broadcasted_iota(jnp.int32, (NT, E), 0)
            eot_ref[...] = jnp.sum((tio >= tilestart.astype(jnp.int32))
                                   .astype(jnp.int32), axis=1,
                                   keepdims=True) - 1


def _mega(src, ln1_g, ln1_b, wq, wk, wv, rel_bias, wo, ln2_g, ln2_b,
          expert_sel):
    rb = jnp.pad(rel_bias, ((0, 0), (0, 4096 - (2 * S - 1))))
    lt = jnp.asarray(np.tril(np.ones((128, 128), np.float32), -1),
                     dtype=jnp.bfloat16)
    sl = jnp.asarray(np.triu(np.ones((E, E), np.float32), 1),
                     dtype=jnp.bfloat16)
    cst = lambda bs: pl.BlockSpec(bs, lambda i, h: tuple(0 for _ in bs))
    spec_1f = pl.BlockSpec((128, 1), lambda i, h: (i, 0))
    return pl.pallas_call(
        _mega_body,
        grid=(NB, H),
        in_specs=[cst((S, D)), cst((1, D)), cst((1, D)),
                  cst((D, D)), cst((D, D)), cst((D, D)),
                  cst((H, 4096)),
                  cst((D, D)), cst((1, D)), cst((1, D)), cst((D, E)),
                  cst((128, 128)), cst((E, E))],
        out_specs=[pl.BlockSpec((128, D), lambda i, h: (i, 0)),
                   pl.BlockSpec((128, D), lambda i, h: (i, 0)),
                   spec_1f, spec_1f, spec_1f, spec_1f, spec_1f, spec_1f,
                   pl.BlockSpec((1, E), lambda i, h: (0, 0)),
                   pl.BlockSpec((NT, 1), lambda i, h: (0, 0))],
        out_shape=[jax.ShapeDtypeStruct((S, D), jnp.float32),
                   jax.ShapeDtypeStruct((S, D), jnp.float32),
                   jax.ShapeDtypeStruct((S, 1), jnp.float32),
                   jax.ShapeDtypeStruct((S, 1), jnp.float32),
                   jax.ShapeDtypeStruct((S, 1), jnp.int32),
                   jax.ShapeDtypeStruct((S, 1), jnp.int32),
                   jax.ShapeDtypeStruct((S, 1), jnp.int32),
                   jax.ShapeDtypeStruct((S, 1), jnp.int32),
                   jax.ShapeDtypeStruct((1, E), jnp.int32),
                   jax.ShapeDtypeStruct((NT, 1), jnp.int32)],
        scratch_shapes=[pltpu.VMEM((ND * H, 128, 128), jnp.bfloat16),
                        pltpu.VMEM((H, S, DH), jnp.bfloat16),
                        pltpu.VMEM((H, S, DH), jnp.bfloat16),
                        pltpu.VMEM((H, S, 2 * DH), jnp.bfloat16),
                        pltpu.VMEM((H, 128, DH), jnp.bfloat16),
                        pltpu.VMEM((1, E), jnp.float32)],
        interpret=_INTERPRET,
    )(src, ln1_g.reshape(1, D), ln1_b.reshape(1, D),
      wq.astype(jnp.bfloat16), wk.astype(jnp.bfloat16),
      wv.astype(jnp.bfloat16), rb, wo.astype(jnp.bfloat16),
      ln2_g.reshape(1, D), ln2_b.reshape(1, D), expert_sel, lt, sl)


def _k7_body(eot_sref, xg_ref, k_ref, v_ref, yg_ref):
    x = xg_ref[...].astype(jnp.bfloat16)
    hid = jax.nn.relu(lax.dot(x, k_ref[0], preferred_element_type=jnp.float32))
    yg_ref[...] = lax.dot(hid.astype(jnp.bfloat16), v_ref[0],
                          preferred_element_type=jnp.float32)


def _k7_group(xg, keys_bf, values_bf, eot_flat):
    grid_spec = pltpu.PrefetchScalarGridSpec(
        num_scalar_prefetch=1,
        grid=(NT,),
        in_specs=[pl.BlockSpec((128, D), lambda t, e: (t, 0)),
                  pl.BlockSpec((1, D, F), lambda t, e: (e[t], 0, 0)),
                  pl.BlockSpec((1, F, D), lambda t, e: (e[t], 0, 0))],
        out_specs=pl.BlockSpec((128, D), lambda t, e: (t, 0)),
    )
    return pl.pallas_call(
        _k7_body,
        grid_spec=grid_spec,
        out_shape=jax.ShapeDtypeStruct((NT * 128, D), jnp.float32),
        interpret=_INTERPRET,
    )(eot_flat, xg, keys_bf, values_bf)


def _sc_mesh():
    return plsc.VectorSubcoreMesh(core_axis_name="c", subcore_axis_name="s",
                                  num_cores=2)


def _k8_dispatch(x3, r0, r1, e0, e1, sp):
    @functools.partial(
        pl.kernel, mesh=_sc_mesh(),
        out_type=[jax.ShapeDtypeStruct((NT * 128, D), jnp.float32),
                  jax.ShapeDtypeStruct((NW, 64), jnp.int32),
                  jax.ShapeDtypeStruct((NW, 64), jnp.int32)],
        scratch_types=[pltpu.VMEM((1, 64), jnp.int32),
                       pltpu.VMEM((1, 64), jnp.int32),
                       pltpu.VMEM((1, 64), jnp.int32),
                       pltpu.VMEM((1, 64), jnp.int32),
                       pltpu.VMEM((1, 64), jnp.int32),
                       pltpu.VMEM((1, 64), jnp.int32),
                       pltpu.VMEM((1, 64), jnp.int32),
                       pltpu.VMEM((64, D), jnp.float32),
                       pltpu.SemaphoreType.DMA],
    )
    def k(x3_hbm, r0_hbm, r1_hbm, e0_hbm, e1_hbm, sp_hbm,
          xg_hbm, p0f_hbm, p1f_hbm,
          spv, r0v, r1v, e0v, e1v, p0v, p1v, rows_v, sem):
        wid = lax.axis_index("s") * 2 + lax.axis_index("c")
        pltpu.sync_copy(sp_hbm.at[pl.ds(0, 1)], spv)
        pltpu.sync_copy(r0_hbm.at[pl.ds(wid, 1)], r0v)
        pltpu.sync_copy(r1_hbm.at[pl.ds(wid, 1)], r1v)
        pltpu.sync_copy(e0_hbm.at[pl.ds(wid, 1)], e0v)
        pltpu.sync_copy(e1_hbm.at[pl.ds(wid, 1)], e1v)
        sp_chunks = [spv[0, pl.ds(k * 16, 16)] for k in range(4)]
        sp_scalar = [sp_chunks[j // 16][j % 16] for j in range(E)]
        for c in range(4):
            slc = pl.ds(c * 16, 16)
            e0c = e0v[0, slc]
            e1c = e1v[0, slc]
            acc0 = jnp.zeros((16,), jnp.int32)
            acc1 = jnp.zeros((16,), jnp.int32)
            for j in range(E):
                acc0 = jnp.where(e0c == j, sp_scalar[j], acc0)
                acc1 = jnp.where(e1c == j, sp_scalar[j], acc1)
            p0v[0, slc] = r0v[0, slc] + acc0
            p1v[0, slc] = r1v[0, slc] + acc1
        pltpu.sync_copy(x3_hbm.at[pl.ds(wid * 64, 64)], rows_v)
        s0 = pltpu.async_copy(rows_v, xg_hbm.at[p0v.at[0]], sem)
        s1 = pltpu.async_copy(rows_v, xg_hbm.at[p1v.at[0]], sem)
        pltpu.sync_copy(p0v, p0f_hbm.at[pl.ds(wid, 1)])
        pltpu.sync_copy(p1v, p1f_hbm.at[pl.ds(wid, 1)])
        s0.wait()
        s1.wait()

    return k(x3, r0, r1, e0, e1, sp)


def _k9_combine(yg, p0, p1, g0, g1, src2):
    @functools.partial(
        pl.kernel, mesh=_sc_mesh(),
        out_type=jax.ShapeDtypeStruct((S, D), jnp.float32),
        scratch_types=[pltpu.VMEM((4, 16), jnp.int32),
                       pltpu.VMEM((4, 16), jnp.int32),
                       pltpu.VMEM((4, 16), jnp.float32),
                       pltpu.VMEM((4, 16), jnp.float32),
                       pltpu.VMEM((16, D), jnp.float32),
                       pltpu.VMEM((16, D), jnp.float32),
                       pltpu.VMEM((16, D), jnp.float32),
                       pltpu.VMEM((16, D), jnp.float32),
                       pltpu.VMEM((16, D), jnp.float32),
                       pltpu.SemaphoreType.DMA,
                       pltpu.SemaphoreType.DMA],
    )
    def k(yg_hbm, p0_hbm, p1_hbm, g0_hbm, g1_hbm, src_hbm, out_hbm,
          p0v, p1v, g0v, g1v, r0a, r1a, r0b, r1b, sv, semA, semB):
        wid = lax.axis_index("s") * 2 + lax.axis_index("c")
        pltpu.sync_copy(p0_hbm.at[pl.ds(wid * 4, 4)], p0v)
        pltpu.sync_copy(p1_hbm.at[pl.ds(wid * 4, 4)], p1v)
        pltpu.sync_copy(g0_hbm.at[pl.ds(wid * 4, 4)], g0v)
        pltpu.sync_copy(g1_hbm.at[pl.ds(wid * 4, 4)], g1v)
        bufs = [(r0a, r1a, semA), (r0b, r1b, semB)]

        def issue(c):
            r0, r1, sm = bufs[c % 2]
            return (pltpu.async_copy(yg_hbm.at[p0v.at[c]], r0, sm),
                    pltpu.async_copy(yg_hbm.at[p1v.at[c]], r1, sm))

        pend = issue(0)
        for c in range(4):
            nxt = issue(c + 1) if c < 3 else None
            base = wid * 64 + c * 16
            pltpu.sync_copy(src_hbm.at[pl.ds(base, 16)], sv)
            g0row = g0v[c, :]
            g1row = g1v[c, :]
            ga = [jnp.full((16,), g0row[i], jnp.float32) for i in range(16)]
            gb = [jnp.full((16,), g1row[i], jnp.float32) for i in range(16)]
            pend[0].wait()
            pend[1].wait()
            r0v, r1v, _ = bufs[c % 2]

            def col(j, _2, r0v=r0v, r1v=r1v, ga=ga, gb=gb):
                sl = pl.ds(j * 16, 16)
                for i in range(16):
                    sv[i, sl] = sv[i, sl] + ga[i] * r0v[i, sl] + gb[i] * r1v[i, sl]
                return 0

            lax.fori_loop(0, D // 16, col, 0)
            pltpu.sync_copy(sv, out_hbm.at[pl.ds(base, 16)])
            pend = nxt

    return k(yg, p0, p1, g0, g1, src2)


def kernel(src, ln1_g, ln1_b, ln2_g, ln2_b, Wq, Wk, Wv, Wo, rel_bias,
           expert_sel, keys, values):
    src2d = src.reshape(S, D)
    (src2, x3, g0, g1, e0, e1, r0, r1, sp, eot) = _mega(
        src2d, ln1_g, ln1_b, Wq, Wk, Wv, rel_bias, Wo, ln2_g, ln2_b,
        expert_sel)
    xg, p0f, p1f = _k8_dispatch(x3, r0.reshape(NW, 64), r1.reshape(NW, 64),
                                e0.reshape(NW, 64), e1.reshape(NW, 64), sp)
    yg = _k7_group(xg, keys.astype(jnp.bfloat16), values.astype(jnp.bfloat16),
                   eot.reshape(NT))
    out = _k9_combine(yg,
                      p0f.reshape(NW * 4, 16),
                      p1f.reshape(NW * 4, 16),
                      g0.reshape(NW * 4, 16),
                      g1.reshape(NW * 4, 16),
                      src2)
    return out.reshape(1, S, D)


# speedup vs baseline: 1.7041x; 1.0134x over previous
import functools

import jax
import jax.numpy as jnp
import numpy as np
from jax import lax
from jax.experimental import pallas as pl
from jax.experimental.pallas import tpu as pltpu
from jax.experimental.pallas import tpu_sc as plsc

S, D, H, E, F = 2048, 768, 12, 64, 64
DH = D // H
NB = S // 128
ND = 2 * NB - 1
NP = 2 * S
NT = 96
NW = 32

_INTERPRET = False


def _pc(body, grid, in_specs, out_specs, out_shape, scratch_shapes=()):
    return pl.pallas_call(
        body,
        grid=grid,
        in_specs=in_specs,
        out_specs=out_specs,
        out_shape=out_shape,
        scratch_shapes=list(scratch_shapes),
        interpret=_INTERPRET,
    )


def _ln(x, g, b):
    m = jnp.mean(x, axis=-1, keepdims=True)
    v = jnp.mean((x - m) ** 2, axis=-1, keepdims=True)
    return (x - m) * jax.lax.rsqrt(v + 1e-5) * g + b


def _mega_body(src_ref, ln1g_ref, ln1b_ref, wq_ref, wk_ref, wv_ref,
               rb_ref, wo_ref, ln2g_ref, ln2b_ref, es_ref,
               lt_ref, sl_ref,
               src2_ref, x3_ref, g0_ref, g1_ref, e0_ref, e1_ref,
               r0_ref, r1_ref, sp_ref, eot_ref,
               t_s, q_s, k_s, v_s, att_s, cnt_s):
    i = pl.program_id(0)
    h = pl.program_id(1)

    @pl.when((i == 0) & (h == 0))
    def _prep():
        rb = rb_ref[...]
        sm = jnp.concatenate(
            [rb[:, 128 * d:128 * d + 256] for d in range(ND)], axis=0
        ).astype(jnp.bfloat16)
        for nc in range(4):
            x = lax.broadcasted_iota(jnp.int32, (256, 4096), 1) + 4096 * nc
            cidx = lax.broadcasted_iota(jnp.int32, (256, 4096), 0)
            e_chunk = (cidx == ((x & 127) - (x >> 7) + 127)).astype(jnp.bfloat16)
            mm = lax.dot(sm, e_chunk, preferred_element_type=jnp.float32)
            t_s[:, 32 * nc:32 * (nc + 1), :] = (
                mm.astype(jnp.bfloat16).reshape(ND * H, 32, 128))
        ones = jnp.ones((128, 1), jnp.bfloat16)
        zeros = jnp.zeros((128, DH - 1), jnp.bfloat16)
        for b in range(NB):
            x2 = _ln(src_ref[pl.ds(128 * b, 128), :], ln1g_ref[...],
                     ln1b_ref[...]).astype(jnp.bfloat16)
            qb = (lax.dot(x2, wq_ref[...], preferred_element_type=jnp.float32)
                  * 0.125).astype(jnp.bfloat16)
            kb = lax.dot(x2, wk_ref[...],
                         preferred_element_type=jnp.float32).astype(jnp.bfloat16)
            vb = lax.dot(x2, wv_ref[...],
                         preferred_element_type=jnp.float32).astype(jnp.bfloat16)
            for hh in range(H):
                cs = slice(hh * DH, (hh + 1) * DH)
                q_s[hh, pl.ds(128 * b, 128), :] = qb[:, cs]
                k_s[hh, pl.ds(128 * b, 128), :] = kb[:, cs]
                v_s[hh, pl.ds(128 * b, 128), :] = jnp.concatenate(
                    [vb[:, cs], ones, zeros], axis=1)

    q = q_s[h, pl.ds(128 * i, 128), :]
    k = k_s[h]
    s = lax.dot_general(q, k, (((1,), (1,)), ((), ())),
                        preferred_element_type=jnp.float32)
    patt = jnp.concatenate(
        [t_s[(j - i + (NB - 1)) * H + h] for j in range(NB)], axis=1)
    m = jnp.max(s, axis=1, keepdims=True) + 0.5
    p = jnp.exp((s - m).astype(jnp.bfloat16) + patt)
    o2 = lax.dot(p, v_s[h], preferred_element_type=jnp.float32)
    att_s[h] = (o2[:, :DH] * (1.0 / o2[:, DH:DH + 1])).astype(jnp.bfloat16)

    @pl.when(h == H - 1)
    def _tail():
        cat = jnp.concatenate([att_s[hh] for hh in range(H)], axis=1)
        o = lax.dot(cat, wo_ref[...], preferred_element_type=jnp.float32)
        src2 = src_ref[pl.ds(128 * i, 128), :] + o
        src2_ref[...] = src2
        x3 = _ln(src2, ln2g_ref[...], ln2b_ref[...])
        x3_ref[...] = x3
        logits = lax.dot(x3, es_ref[...], preferred_element_type=jnp.float32)
        sel = jax.nn.sigmoid(logits)
        iota = lax.broadcasted_iota(jnp.int32, sel.shape, 1)
        m1 = jnp.max(sel, axis=1, keepdims=True)
        i1 = jnp.min(jnp.where(sel == m1, iota, E), axis=1, keepdims=True)
        masked = jnp.where(iota == i1, -1.0, sel)
        m2 = jnp.max(masked, axis=1, keepdims=True)
        i2 = jnp.min(jnp.where(masked == m2, iota, E), axis=1, keepdims=True)
        g0_ref[...] = m1
        g1_ref[...] = m2
        e0_ref[...] = i1
        e1_ref[...] = i2
        cnt = jnp.where(i == 0, jnp.zeros((1, E), jnp.float32), cnt_s[...])
        oh0 = (i1 == iota).astype(jnp.float32)
        oh1 = (i2 == iota).astype(jnp.float32)
        ohs = oh0 + oh1
        cums = lax.dot(lt_ref[...], ohs.astype(jnp.bfloat16),
                       preferred_element_type=jnp.float32)
        r0_ref[...] = jnp.sum(oh0 * (cums + cnt), axis=1,
                              keepdims=True).astype(jnp.int32)
        r1_ref[...] = jnp.sum(oh1 * (cums + cnt + oh0), axis=1,
                              keepdims=True).astype(jnp.int32)
        newcnt = cnt + jnp.sum(ohs, axis=0, keepdims=True)
        cnt_s[...] = newcnt

        @pl.when(i == NB - 1)
        def _finalize():
            ntiles = jnp.ceil(newcnt * (1.0 / 128.0))
            tilestart = lax.dot(ntiles.astype(jnp.bfloat16), sl_ref[...],
                                preferred_element_type=jnp.float32)
            sp_ref[...] = (tilestart * 128.0).astype(jnp.int32)
            tio = lax.broadcasted_iota(jnp.int32, (NT, E), 0)
            eot = jnp.sum((tio >= tilestart.astype(jnp.int32))
                          .astype(jnp.int32), axis=1, keepdims=True) - 1
            tot = jnp.sum(ntiles).astype(jnp.int32)
            tio0 = lax.broadcasted_iota(jnp.int32, (NT, 1), 0)
            eot_ref[...] = eot + jnp.where(tio0 >= tot, E, 0)


def _mega(src, ln1_g, ln1_b, wq, wk, wv, rel_bias, wo, ln2_g, ln2_b,
          expert_sel):
    rb = jnp.pad(rel_bias, ((0, 0), (0, 4096 - (2 * S - 1))))
    lt = jnp.asarray(np.tril(np.ones((128, 128), np.float32), -1),
                     dtype=jnp.bfloat16)
    sl = jnp.asarray(np.triu(np.ones((E, E), np.float32), 1),
                     dtype=jnp.bfloat16)
    cst = lambda bs: pl.BlockSpec(bs, lambda i, h: tuple(0 for _ in bs))
    spec_1f = pl.BlockSpec((128, 1), lambda i, h: (i, 0))
    return pl.pallas_call(
        _mega_body,
        grid=(NB, H),
        in_specs=[cst((S, D)), cst((1, D)), cst((1, D)),
                  cst((D, D)), cst((D, D)), cst((D, D)),
                  cst((H, 4096)),
                  cst((D, D)), cst((1, D)), cst((1, D)), cst((D, E)),
                  cst((128, 128)), cst((E, E))],
        out_specs=[pl.BlockSpec((128, D), lambda i, h: (i, 0)),
                   pl.BlockSpec((128, D), lambda i, h: (i, 0)),
                   spec_1f, spec_1f, spec_1f, spec_1f, spec_1f, spec_1f,
                   pl.BlockSpec((1, E), lambda i, h: (0, 0)),
                   pl.BlockSpec((NT, 1), lambda i, h: (0, 0))],
        out_shape=[jax.ShapeDtypeStruct((S, D), jnp.float32),
                   jax.ShapeDtypeStruct((S, D), jnp.float32),
                   jax.ShapeDtypeStruct((S, 1), jnp.float32),
                   jax.ShapeDtypeStruct((S, 1), jnp.float32),
                   jax.ShapeDtypeStruct((S, 1), jnp.int32),
                   jax.ShapeDtypeStruct((S, 1), jnp.int32),
                   jax.ShapeDtypeStruct((S, 1), jnp.int32),
                   jax.ShapeDtypeStruct((S, 1), jnp.int32),
                   jax.ShapeDtypeStruct((1, E), jnp.int32),
                   jax.ShapeDtypeStruct((NT, 1), jnp.int32)],
        scratch_shapes=[pltpu.VMEM((ND * H, 128, 128), jnp.bfloat16),
                        pltpu.VMEM((H, S, DH), jnp.bfloat16),
                        pltpu.VMEM((H, S, DH), jnp.bfloat16),
                        pltpu.VMEM((H, S, 2 * DH), jnp.bfloat16),
                        pltpu.VMEM((H, 128, DH), jnp.bfloat16),
                        pltpu.VMEM((1, E), jnp.float32)],
        interpret=_INTERPRET,
    )(src, ln1_g.reshape(1, D), ln1_b.reshape(1, D),
      wq.astype(jnp.bfloat16), wk.astype(jnp.bfloat16),
      wv.astype(jnp.bfloat16), rb, wo.astype(jnp.bfloat16),
      ln2_g.reshape(1, D), ln2_b.reshape(1, D), expert_sel, lt, sl)


def _k7_body(eot_sref, xg_ref, k_ref, v_ref, yg_ref):
    t = pl.program_id(0)

    @pl.when(eot_sref[t] < E)
    def _active():
        x = xg_ref[...].astype(jnp.bfloat16)
        hid = jax.nn.relu(lax.dot(x, k_ref[0],
                                  preferred_element_type=jnp.float32))
        yg_ref[...] = lax.dot(hid.astype(jnp.bfloat16), v_ref[0],
                              preferred_element_type=jnp.float32)


def _k7_group(xg, keys_bf, values_bf, eot_flat):
    grid_spec = pltpu.PrefetchScalarGridSpec(
        num_scalar_prefetch=1,
        grid=(NT,),
        in_specs=[pl.BlockSpec((128, D), lambda t, e: (t, 0)),
                  pl.BlockSpec((1, D, F), lambda t, e: (e[t] % E, 0, 0)),
                  pl.BlockSpec((1, F, D), lambda t, e: (e[t] % E, 0, 0))],
        out_specs=pl.BlockSpec((128, D), lambda t, e: (t, 0)),
    )
    return pl.pallas_call(
        _k7_body,
        grid_spec=grid_spec,
        out_shape=jax.ShapeDtypeStruct((NT * 128, D), jnp.float32),
        interpret=_INTERPRET,
    )(eot_flat, xg, keys_bf, values_bf)


def _sc_mesh():
    return plsc.VectorSubcoreMesh(core_axis_name="c", subcore_axis_name="s",
                                  num_cores=2)


def _k8_dispatch(x3, r0, r1, e0, e1, sp):
    @functools.partial(
        pl.kernel, mesh=_sc_mesh(),
        out_type=[jax.ShapeDtypeStruct((NT * 128, D), jnp.float32),
                  jax.ShapeDtypeStruct((NW, 64), jnp.int32),
                  jax.ShapeDtypeStruct((NW, 64), jnp.int32)],
        scratch_types=[pltpu.VMEM((1, 64), jnp.int32),
                       pltpu.VMEM((1, 64), jnp.int32),
                       pltpu.VMEM((1, 64), jnp.int32),
                       pltpu.VMEM((1, 64), jnp.int32),
                       pltpu.VMEM((1, 64), jnp.int32),
                       pltpu.VMEM((1, 64), jnp.int32),
                       pltpu.VMEM((1, 64), jnp.int32),
                       pltpu.VMEM((64, D), jnp.float32),
                       pltpu.SemaphoreType.DMA],
    )
    def k(x3_hbm, r0_hbm, r1_hbm, e0_hbm, e1_hbm, sp_hbm,
          xg_hbm, p0f_hbm, p1f_hbm,
          spv, r0v, r1v, e0v, e1v, p0v, p1v, rows_v, sem):
        wid = lax.axis_index("s") * 2 + lax.axis_index("c")
        pltpu.sync_copy(sp_hbm.at[pl.ds(0, 1)], spv)
        pltpu.sync_copy(r0_hbm.at[pl.ds(wid, 1)], r0v)
        pltpu.sync_copy(r1_hbm.at[pl.ds(wid, 1)], r1v)
        pltpu.sync_copy(e0_hbm.at[pl.ds(wid, 1)], e0v)
        pltpu.sync_copy(e1_hbm.at[pl.ds(wid, 1)], e1v)
        sp_chunks = [spv[0, pl.ds(k * 16, 16)] for k in range(4)]
        sp_scalar = [sp_chunks[j // 16][j % 16] for j in range(E)]
        for c in range(4):
            slc = pl.ds(c * 16, 16)
            e0c = e0v[0, slc]
            e1c = e1v[0, slc]
            acc0 = jnp.zeros((16,), jnp.int32)
            acc1 = jnp.zeros((16,), jnp.int32)
            for j in range(E):
                acc0 = jnp.where(e0c == j, sp_scalar[j], acc0)
                acc1 = jnp.where(e1c == j, sp_scalar[j], acc1)
            p0v[0, slc] = r0v[0, slc] + acc0
            p1v[0, slc] = r1v[0, slc] + acc1
        pltpu.sync_copy(x3_hbm.at[pl.ds(wid * 64, 64)], rows_v)
        s0 = pltpu.async_copy(rows_v, xg_hbm.at[p0v.at[0]], sem)
        s1 = pltpu.async_copy(rows_v, xg_hbm.at[p1v.at[0]], sem)
        pltpu.sync_copy(p0v, p0f_hbm.at[pl.ds(wid, 1)])
        pltpu.sync_copy(p1v, p1f_hbm.at[pl.ds(wid, 1)])
        s0.wait()
        s1.wait()

    return k(x3, r0, r1, e0, e1, sp)


def _k9_combine(yg, p0, p1, g0, g1, src2):
    @functools.partial(
        pl.kernel, mesh=_sc_mesh(),
        out_type=jax.ShapeDtypeStruct((S, D), jnp.float32),
        scratch_types=[pltpu.VMEM((4, 16), jnp.int32),
                       pltpu.VMEM((4, 16), jnp.int32),
                       pltpu.VMEM((4, 16), jnp.float32),
                       pltpu.VMEM((4, 16), jnp.float32),
                       pltpu.VMEM((16, D), jnp.float32),
                       pltpu.VMEM((16, D), jnp.float32),
                       pltpu.VMEM((16, D), jnp.float32),
                       pltpu.VMEM((16, D), jnp.float32),
                       pltpu.VMEM((16, D), jnp.float32),
                       pltpu.SemaphoreType.DMA,
                       pltpu.SemaphoreType.DMA],
    )
    def k(yg_hbm, p0_hbm, p1_hbm, g0_hbm, g1_hbm, src_hbm, out_hbm,
          p0v, p1v, g0v, g1v, r0a, r1a, r0b, r1b, sv, semA, semB):
        wid = lax.axis_index("s") * 2 + lax.axis_index("c")
        pltpu.sync_copy(p0_hbm.at[pl.ds(wid * 4, 4)], p0v)
        pltpu.sync_copy(p1_hbm.at[pl.ds(wid * 4, 4)], p1v)
        pltpu.sync_copy(g0_hbm.at[pl.ds(wid * 4, 4)], g0v)
        pltpu.sync_copy(g1_hbm.at[pl.ds(wid * 4, 4)], g1v)
        bufs = [(r0a, r1a, semA), (r0b, r1b, semB)]

        def issue(c):
            r0, r1, sm = bufs[c % 2]
            return (pltpu.async_copy(yg_hbm.at[p0v.at[c]], r0, sm),
                    pltpu.async_copy(yg_hbm.at[p1v.at[c]], r1, sm))

        pend = issue(0)
        for c in range(4):
            nxt = issue(c + 1) if c < 3 else None
            base = wid * 64 + c * 16
            pltpu.sync_copy(src_hbm.at[pl.ds(base, 16)], sv)
            g0row = g0v[c, :]
            g1row = g1v[c, :]
            ga = [jnp.full((16,), g0row[i], jnp.float32) for i in range(16)]
            gb = [jnp.full((16,), g1row[i], jnp.float32) for i in range(16)]
            pend[0].wait()
            pend[1].wait()
            r0v, r1v, _ = bufs[c % 2]

            def col(j, _2, r0v=r0v, r1v=r1v, ga=ga, gb=gb):
                sl = pl.ds(j * 16, 16)
                for i in range(16):
                    sv[i, sl] = sv[i, sl] + ga[i] * r0v[i, sl] + gb[i] * r1v[i, sl]
                return 0

            lax.fori_loop(0, D // 16, col, 0)
            pltpu.sync_copy(sv, out_hbm.at[pl.ds(base, 16)])
            pend = nxt

    return k(yg, p0, p1, g0, g1, src2)


def kernel(src, ln1_g, ln1_b, ln2_g, ln2_b, Wq, Wk, Wv, Wo, rel_bias,
           expert_sel, keys, values):
    src2d = src.reshape(S, D)
    (src2, x3, g0, g1, e0, e1, r0, r1, sp, eot) = _mega(
        src2d, ln1_g, ln1_b, Wq, Wk, Wv, rel_bias, Wo, ln2_g, ln2_b,
        expert_sel)
    xg, p0f, p1f = _k8_dispatch(x3, r0.reshape(NW, 64), r1.reshape(NW, 64),
                                e0.reshape(NW, 64), e1.reshape(NW, 64), sp)
    yg = _k7_group(xg, keys.astype(jnp.bfloat16), values.astype(jnp.bfloat16),
                   eot.reshape(NT))
    out = _k9_combine(yg,
                      p0f.reshape(NW * 4, 16),
                      p1f.reshape(NW * 4, 16),
                      g0.reshape(NW * 4, 16),
                      g1.reshape(NW * 4, 16),
                      src2)
    return out.reshape(1, S, D)


# speedup vs baseline: 1.7047x; 1.0003x over previous
import functools

import jax
import jax.numpy as jnp
import numpy as np
from jax import lax
from jax.experimental import pallas as pl
from jax.experimental.pallas import tpu as pltpu
from jax.experimental.pallas import tpu_sc as plsc

S, D, H, E, F = 2048, 768, 12, 64, 64
DH = D // H
NB = S // 128
ND = 2 * NB - 1
NP = 2 * S
NT = 96
NW = 32


def _ln(x, g, b):
    m = jnp.mean(x, axis=-1, keepdims=True)
    v = jnp.mean((x - m) ** 2, axis=-1, keepdims=True)
    return (x - m) * jax.lax.rsqrt(v + 1e-5) * g + b


def _mega_body(src_ref, ln1g_ref, ln1b_ref, wq_ref, wk_ref, wv_ref,
               rb_ref, wo_ref, ln2g_ref, ln2b_ref, es_ref,
               lt_ref, sl_ref,
               src2_ref, x3_ref, g0_ref, g1_ref, e0_ref, e1_ref,
               r0_ref, r1_ref, sp_ref, eot_ref,
               t_s, q_s, k_s, v_s, att_s, cnt_s):
    i = pl.program_id(0)
    h = pl.program_id(1)

    @pl.when((i == 0) & (h == 0))
    def _prep():
        rb = rb_ref[...]
        sm = jnp.concatenate(
            [rb[:, 128 * d:128 * d + 256] for d in range(ND)], axis=0
        ).astype(jnp.bfloat16)
        for nc in range(4):
            x = lax.broadcasted_iota(jnp.int32, (256, 4096), 1) + 4096 * nc
            cidx = lax.broadcasted_iota(jnp.int32, (256, 4096), 0)
            e_chunk = (cidx == ((x & 127) - (x >> 7) + 127)).astype(jnp.bfloat16)
            mm = lax.dot(sm, e_chunk, preferred_element_type=jnp.float32)
            t_s[:, 32 * nc:32 * (nc + 1), :] = (
                mm.astype(jnp.bfloat16).reshape(ND * H, 32, 128))
        ones = jnp.ones((128, 1), jnp.bfloat16)
        zeros = jnp.zeros((128, DH - 1), jnp.bfloat16)
        for b in range(NB):
            x2 = _ln(src_ref[pl.ds(128 * b, 128), :], ln1g_ref[...],
                     ln1b_ref[...]).astype(jnp.bfloat16)
            qb = (lax.dot(x2, wq_ref[...], preferred_element_type=jnp.float32)
                  * 0.125).astype(jnp.bfloat16)
            kb = lax.dot(x2, wk_ref[...],
                         preferred_element_type=jnp.float32).astype(jnp.bfloat16)
            vb = lax.dot(x2, wv_ref[...],
                         preferred_element_type=jnp.float32).astype(jnp.bfloat16)
            for hh in range(H):
                cs = slice(hh * DH, (hh + 1) * DH)
                q_s[hh, pl.ds(128 * b, 128), :] = qb[:, cs]
                k_s[hh, pl.ds(128 * b, 128), :] = kb[:, cs]
                v_s[hh, pl.ds(128 * b, 128), :] = jnp.concatenate(
                    [vb[:, cs], ones, zeros], axis=1)

    q = q_s[h, pl.ds(128 * i, 128), :]
    k = k_s[h]
    s = lax.dot_general(q, k, (((1,), (1,)), ((), ())),
                        preferred_element_type=jnp.float32)
    patt = jnp.concatenate(
        [t_s[(j - i + (NB - 1)) * H + h] for j in range(NB)], axis=1)
    m = jnp.max(s, axis=1, keepdims=True) + 0.5
    p = jnp.exp((s - m).astype(jnp.bfloat16) + patt)
    o2 = lax.dot(p, v_s[h], preferred_element_type=jnp.float32)
    att_s[h] = (o2[:, :DH] * (1.0 / o2[:, DH:DH + 1])).astype(jnp.bfloat16)

    @pl.when(h == H - 1)
    def _tail():
        cat = jnp.concatenate([att_s[hh] for hh in range(H)], axis=1)
        o = lax.dot(cat, wo_ref[...], preferred_element_type=jnp.float32)
        src2 = src_ref[pl.ds(128 * i, 128), :] + o
        src2_ref[...] = src2
        x3 = _ln(src2, ln2g_ref[...], ln2b_ref[...])
        x3_ref[...] = x3
        logits = lax.dot(x3, es_ref[...], preferred_element_type=jnp.float32)
        sel = jax.nn.sigmoid(logits)
        iota = lax.broadcasted_iota(jnp.int32, sel.shape, 1)
        m1 = jnp.max(sel, axis=1, keepdims=True)
        i1 = jnp.min(jnp.where(sel == m1, iota, E), axis=1, keepdims=True)
        masked = jnp.where(iota == i1, -1.0, sel)
        m2 = jnp.max(masked, axis=1, keepdims=True)
        i2 = jnp.min(jnp.where(masked == m2, iota, E), axis=1, keepdims=True)
        g0_ref[...] = m1
        g1_ref[...] = m2
        e0_ref[...] = i1
        e1_ref[...] = i2
        cnt = jnp.where(i == 0, jnp.zeros((1, E), jnp.float32), cnt_s[...])
        oh0 = (i1 == iota).astype(jnp.float32)
        oh1 = (i2 == iota).astype(jnp.float32)
        ohs = oh0 + oh1
        cums = lax.dot(lt_ref[...], ohs.astype(jnp.bfloat16),
                       preferred_element_type=jnp.float32)
        r0_ref[...] = jnp.sum(oh0 * (cums + cnt), axis=1,
                              keepdims=True).astype(jnp.int32)
        r1_ref[...] = jnp.sum(oh1 * (cums + cnt + oh0), axis=1,
                              keepdims=True).astype(jnp.int32)
        newcnt = cnt + jnp.sum(ohs, axis=0, keepdims=True)
        cnt_s[...] = newcnt

        @pl.when(i == NB - 1)
        def _finalize():
            ntiles = jnp.ceil(newcnt * (1.0 / 128.0))
            tilestart = lax.dot(ntiles.astype(jnp.bfloat16), sl_ref[...],
                                preferred_element_type=jnp.float32)
            sp_ref[...] = (tilestart * 128.0).astype(jnp.int32)
            tio = lax.broadcasted_iota(jnp.int32, (NT, E), 0)
            eot = jnp.sum((tio >= tilestart.astype(jnp.int32))
                          .astype(jnp.int32), axis=1, keepdims=True) - 1
            tot = jnp.sum(ntiles).astype(jnp.int32)
            tio0 = lax.broadcasted_iota(jnp.int32, (NT, 1), 0)
            eot_ref[...] = eot + jnp.where(tio0 >= tot, E, 0)


def _mega(src, ln1_g, ln1_b, wq, wk, wv, rel_bias, wo, ln2_g, ln2_b,
          expert_sel):
    rb = jnp.pad(rel_bias, ((0, 0), (0, 4096 - (2 * S - 1))))
    lt = jnp.asarray(np.tril(np.ones((128, 128), np.float32), -1),
                     dtype=jnp.bfloat16)
    sl = jnp.asarray(np.triu(np.ones((E, E), np.float32), 1),
                     dtype=jnp.bfloat16)
    cst = lambda bs: pl.BlockSpec(bs, lambda i, h: tuple(0 for _ in bs))
    spec_1f = pl.BlockSpec((128, 1), lambda i, h: (i, 0))
    return pl.pallas_call(
        _mega_body,
        grid=(NB, H),
        in_specs=[cst((S, D)), cst((1, D)), cst((1, D)),
                  cst((D, D)), cst((D, D)), cst((D, D)),
                  cst((H, 4096)),
                  cst((D, D)), cst((1, D)), cst((1, D)), cst((D, E)),
                  cst((128, 128)), cst((E, E))],
        out_specs=[pl.BlockSpec((128, D), lambda i, h: (i, 0)),
                   pl.BlockSpec((128, D), lambda i, h: (i, 0)),
                   spec_1f, spec_1f, spec_1f, spec_1f, spec_1f, spec_1f,
                   pl.BlockSpec((1, E), lambda i, h: (0, 0)),
                   pl.BlockSpec((NT, 1), lambda i, h: (0, 0))],
        out_shape=[jax.ShapeDtypeStruct((S, D), jnp.float32),
                   jax.ShapeDtypeStruct((S, D), jnp.float32),
                   jax.ShapeDtypeStruct((S, 1), jnp.float32),
                   jax.ShapeDtypeStruct((S, 1), jnp.float32),
                   jax.ShapeDtypeStruct((S, 1), jnp.int32),
                   jax.ShapeDtypeStruct((S, 1), jnp.int32),
                   jax.ShapeDtypeStruct((S, 1), jnp.int32),
                   jax.ShapeDtypeStruct((S, 1), jnp.int32),
                   jax.ShapeDtypeStruct((1, E), jnp.int32),
                   jax.ShapeDtypeStruct((NT, 1), jnp.int32)],
        scratch_shapes=[pltpu.VMEM((ND * H, 128, 128), jnp.bfloat16),
                        pltpu.VMEM((H, S, DH), jnp.bfloat16),
                        pltpu.VMEM((H, S, DH), jnp.bfloat16),
                        pltpu.VMEM((H, S, 2 * DH), jnp.bfloat16),
                        pltpu.VMEM((H, 128, DH), jnp.bfloat16),
                        pltpu.VMEM((1, E), jnp.float32)],
    )(src, ln1_g.reshape(1, D), ln1_b.reshape(1, D),
      wq.astype(jnp.bfloat16), wk.astype(jnp.bfloat16),
      wv.astype(jnp.bfloat16), rb, wo.astype(jnp.bfloat16),
      ln2_g.reshape(1, D), ln2_b.reshape(1, D), expert_sel, lt, sl)


def _k7_body(eot_sref, xg_ref, k_ref, v_ref, yg_ref):
    t = pl.program_id(0)

    @pl.when(eot_sref[t] < E)
    def _active():
        x = xg_ref[...].astype(jnp.bfloat16)
        hid = jax.nn.relu(lax.dot(x, k_ref[0],
                                  preferred_element_type=jnp.float32))
        yg_ref[...] = lax.dot(hid.astype(jnp.bfloat16), v_ref[0],
                              preferred_element_type=jnp.float32)


def _k7_group(xg, keys_bf, values_bf, eot_flat):
    grid_spec = pltpu.PrefetchScalarGridSpec(
        num_scalar_prefetch=1,
        grid=(NT,),
        in_specs=[pl.BlockSpec((128, D), lambda t, e: (t, 0)),
                  pl.BlockSpec((1, D, F), lambda t, e: (e[t] % E, 0, 0)),
                  pl.BlockSpec((1, F, D), lambda t, e: (e[t] % E, 0, 0))],
        out_specs=pl.BlockSpec((128, D), lambda t, e: (t, 0)),
    )
    return pl.pallas_call(
        _k7_body,
        grid_spec=grid_spec,
        out_shape=jax.ShapeDtypeStruct((NT * 128, D), jnp.float32),
    )(eot_flat, xg, keys_bf, values_bf)


def _sc_mesh():
    return plsc.VectorSubcoreMesh(core_axis_name="c", subcore_axis_name="s",
                                  num_cores=2)


def _k8_dispatch(x3, r0, r1, e0, e1, sp):
    @functools.partial(
        pl.kernel, mesh=_sc_mesh(),
        out_type=[jax.ShapeDtypeStruct((NT * 128, D), jnp.float32),
                  jax.ShapeDtypeStruct((NW, 64), jnp.int32),
                  jax.ShapeDtypeStruct((NW, 64), jnp.int32)],
        scratch_types=[pltpu.VMEM((1, 64), jnp.int32),
                       pltpu.VMEM((1, 64), jnp.int32),
                       pltpu.VMEM((1, 64), jnp.int32),
                       pltpu.VMEM((1, 64), jnp.int32),
                       pltpu.VMEM((1, 64), jnp.int32),
                       pltpu.VMEM((1, 64), jnp.int32),
                       pltpu.VMEM((1, 64), jnp.int32),
                       pltpu.VMEM((64, D), jnp.float32),
                       pltpu.SemaphoreType.DMA],
    )
    def k(x3_hbm, r0_hbm, r1_hbm, e0_hbm, e1_hbm, sp_hbm,
          xg_hbm, p0f_hbm, p1f_hbm,
          spv, r0v, r1v, e0v, e1v, p0v, p1v, rows_v, sem):
        wid = lax.axis_index("s") * 2 + lax.axis_index("c")
        pltpu.sync_copy(sp_hbm.at[pl.ds(0, 1)], spv)
        pltpu.sync_copy(r0_hbm.at[pl.ds(wid, 1)], r0v)
        pltpu.sync_copy(r1_hbm.at[pl.ds(wid, 1)], r1v)
        pltpu.sync_copy(e0_hbm.at[pl.ds(wid, 1)], e0v)
        pltpu.sync_copy(e1_hbm.at[pl.ds(wid, 1)], e1v)
        sp_chunks = [spv[0, pl.ds(k * 16, 16)] for k in range(4)]
        sp_scalar = [sp_chunks[j // 16][j % 16] for j in range(E)]
        for c in range(4):
            slc = pl.ds(c * 16, 16)
            e0c = e0v[0, slc]
            e1c = e1v[0, slc]
            acc0 = jnp.zeros((16,), jnp.int32)
            acc1 = jnp.zeros((16,), jnp.int32)
            for j in range(E):
                acc0 = jnp.where(e0c == j, sp_scalar[j], acc0)
                acc1 = jnp.where(e1c == j, sp_scalar[j], acc1)
            p0v[0, slc] = r0v[0, slc] + acc0
            p1v[0, slc] = r1v[0, slc] + acc1
        pltpu.sync_copy(x3_hbm.at[pl.ds(wid * 64, 64)], rows_v)
        s0 = pltpu.async_copy(rows_v, xg_hbm.at[p0v.at[0]], sem)
        s1 = pltpu.async_copy(rows_v, xg_hbm.at[p1v.at[0]], sem)
        pltpu.sync_copy(p0v, p0f_hbm.at[pl.ds(wid, 1)])
        pltpu.sync_copy(p1v, p1f_hbm.at[pl.ds(wid, 1)])
        s0.wait()
        s1.wait()

    return k(x3, r0, r1, e0, e1, sp)


def _k9_combine(yg, p0, p1, g0, g1, src2):
    @functools.partial(
        pl.kernel, mesh=_sc_mesh(),
        out_type=jax.ShapeDtypeStruct((S, D), jnp.float32),
        scratch_types=[pltpu.VMEM((4, 16), jnp.int32),
                       pltpu.VMEM((4, 16), jnp.int32),
                       pltpu.VMEM((4, 16), jnp.float32),
                       pltpu.VMEM((4, 16), jnp.float32),
                       pltpu.VMEM((16, D), jnp.float32),
                       pltpu.VMEM((16, D), jnp.float32),
                       pltpu.VMEM((16, D), jnp.float32),
                       pltpu.VMEM((16, D), jnp.float32),
                       pltpu.VMEM((16, D), jnp.float32),
                       pltpu.SemaphoreType.DMA,
                       pltpu.SemaphoreType.DMA],
    )
    def k(yg_hbm, p0_hbm, p1_hbm, g0_hbm, g1_hbm, src_hbm, out_hbm,
          p0v, p1v, g0v, g1v, r0a, r1a, r0b, r1b, sv, semA, semB):
        wid = lax.axis_index("s") * 2 + lax.axis_index("c")
        pltpu.sync_copy(p0_hbm.at[pl.ds(wid * 4, 4)], p0v)
        pltpu.sync_copy(p1_hbm.at[pl.ds(wid * 4, 4)], p1v)
        pltpu.sync_copy(g0_hbm.at[pl.ds(wid * 4, 4)], g0v)
        pltpu.sync_copy(g1_hbm.at[pl.ds(wid * 4, 4)], g1v)
        bufs = [(r0a, r1a, semA), (r0b, r1b, semB)]

        def issue(c):
            r0, r1, sm = bufs[c % 2]
            return (pltpu.async_copy(yg_hbm.at[p0v.at[c]], r0, sm),
                    pltpu.async_copy(yg_hbm.at[p1v.at[c]], r1, sm))

        pend = issue(0)
        for c in range(4):
            nxt = issue(c + 1) if c < 3 else None
            base = wid * 64 + c * 16
            pltpu.sync_copy(src_hbm.at[pl.ds(base, 16)], sv)
            g0row = g0v[c, :]
            g1row = g1v[c, :]
            ga = [jnp.full((16,), g0row[i], jnp.float32) for i in range(16)]
            gb = [jnp.full((16,), g1row[i], jnp.float32) for i in range(16)]
            pend[0].wait()
            pend[1].wait()
            r0v, r1v, _ = bufs[c % 2]

            def col(j, _2, r0v=r0v, r1v=r1v, ga=ga, gb=gb):
                sl = pl.ds(j * 16, 16)
                for i in range(16):
                    sv[i, sl] = sv[i, sl] + ga[i] * r0v[i, sl] + gb[i] * r1v[i, sl]
                return 0

            lax.fori_loop(0, D // 16, col, 0)
            pltpu.sync_copy(sv, out_hbm.at[pl.ds(base, 16)])
            pend = nxt

    return k(yg, p0, p1, g0, g1, src2)


def kernel(src, ln1_g, ln1_b, ln2_g, ln2_b, Wq, Wk, Wv, Wo, rel_bias,
           expert_sel, keys, values):
    src2d = src.reshape(S, D)
    (src2, x3, g0, g1, e0, e1, r0, r1, sp, eot) = _mega(
        src2d, ln1_g, ln1_b, Wq, Wk, Wv, rel_bias, Wo, ln2_g, ln2_b,
        expert_sel)
    xg, p0f, p1f = _k8_dispatch(x3, r0.reshape(NW, 64), r1.reshape(NW, 64),
                                e0.reshape(NW, 64), e1.reshape(NW, 64), sp)
    yg = _k7_group(xg, keys.astype(jnp.bfloat16), values.astype(jnp.bfloat16),
                   eot.reshape(NT))
    out = _k9_combine(yg,
                      p0f.reshape(NW * 4, 16),
                      p1f.reshape(NW * 4, 16),
                      g0.reshape(NW * 4, 16),
                      g1.reshape(NW * 4, 16),
                      src2)
    return out.reshape(1, S, D)
